# Initial kernel scaffold; baseline (speedup 1.0000x reference)
import jax
import jax.numpy as jnp
from jax.experimental import pallas as pl


def kernel(edge_index, features, W1, b1, W2, b2):
    raise NotImplementedError("write your pallas kernel here")


# speedup vs baseline: 3.3060x; 3.3060x over previous
import functools
import jax
import jax.numpy as jnp
from jax import lax
from jax.experimental import pallas as pl
from jax.experimental.pallas import tpu as pltpu
from jax.experimental.pallas import tpu_sc as plsc

_N = 10000
_E = 320000
_D = 128
_H = 64
_C = 40
_CP = 48
_TILES = 32
_EPT = _E // _TILES
_CH = 80
_NCH = _EPT // _CH
_RPT = 1000

_EPS = _E // 16
_TWS = 20
_TW = 1 << _TWS
_NB = 96
_GRP = 12
_NGRP = _NB // _GRP
_BCAP = 32
_BSTRIDE = 16 * _BCAP
_TSIZE = _TW + 512
_DEGSZ = 11264

_mesh = plsc.VectorSubcoreMesh(core_axis_name="c", subcore_axis_name="s")
_scparams = pltpu.CompilerParams(needs_layout_passes=False,
                                 use_tc_tiling_on_sc=False)


def _make_dots():
  @functools.partial(
      pl.kernel, mesh=_mesh, compiler_params=_scparams,
      out_type=jax.ShapeDtypeStruct((_E,), jnp.float32),
      scratch_types=[
          pltpu.VMEM((_EPT,), jnp.float32),
          pltpu.VMEM((_CH, _D), jnp.float32),
          pltpu.VMEM((_CH, _D), jnp.float32),
          pltpu.VMEM((_CH,), jnp.int32),
          pltpu.VMEM((_CH,), jnp.int32),
          pltpu.VMEM((256,), jnp.float32),
          pltpu.SemaphoreType.DMA,
          pltpu.SemaphoreType.DMA,
      ],
  )
  def dots(src_hbm, dst_hbm, feat_hbm, w_out,
           wown, Xs, Xt, srcv, dstv, tsc, sem1, sem2):
    c = lax.axis_index("c")
    s = lax.axis_index("s")
    g = s * 2 + c
    iota = lax.iota(jnp.int32, 16)

    def chunk(t, _):
      base = g * _EPT + t * _CH
      pltpu.sync_copy(src_hbm.at[pl.ds(base, _CH)], srcv)
      pltpu.sync_copy(dst_hbm.at[pl.ds(base, _CH)], dstv)
      cp1 = pltpu.async_copy(feat_hbm.at[srcv], Xs, sem1)
      cp2 = pltpu.async_copy(feat_hbm.at[dstv], Xt, sem2)
      cp1.wait()
      cp2.wait()
      for grp in range(_CH // 16):
        for j in range(16):
          e = grp * 16 + j
          acc = Xs[e, pl.ds(0, 16)] * Xt[e, pl.ds(0, 16)]
          for i in range(1, _D // 16):
            acc = acc + Xs[e, pl.ds(i * 16, 16)] * Xt[e, pl.ds(i * 16, 16)]
          tsc[pl.ds(j * 16, 16)] = acc
        dotv = plsc.load_gather(tsc, [iota * 16])
        for col in range(1, 16):
          dotv = dotv + plsc.load_gather(tsc, [iota * 16 + col])
        sv = srcv[pl.ds(grp * 16, 16)]
        dv = dstv[pl.ds(grp * 16, 16)]
        keep = (dotv >= 0.0) & (sv != dv)
        wown[pl.ds(t * _CH + grp * 16, 16)] = jnp.where(keep, 1.0, 0.0)
      return 0
    lax.fori_loop(0, _NCH, chunk, 0)
    pltpu.sync_copy(wown, w_out.at[pl.ds(g * _EPT, _EPT)])
  return dots


def _make_count():
  @functools.partial(
      pl.kernel, mesh=_mesh, compiler_params=_scparams,
      out_type=[
          jax.ShapeDtypeStruct((_E,), jnp.float32),
          jax.ShapeDtypeStruct((2, _N), jnp.float32),
          jax.ShapeDtypeStruct((2, _N), jnp.float32),
      ],
      scratch_types=[
          pltpu.VMEM((_EPS,), jnp.int32),
          pltpu.VMEM(((_GRP + 1) * _BSTRIDE,), jnp.int32),
          pltpu.VMEM(((_GRP + 1) * 16,), jnp.int32),
          pltpu.VMEM((_EPT + 16,), jnp.float32),
          pltpu.VMEM((_CH,), jnp.int32),
          pltpu.VMEM((_CH,), jnp.int32),
          pltpu.VMEM((_BSTRIDE,), jnp.int32),
          pltpu.VMEM((_BSTRIDE,), jnp.int32),
          pltpu.VMEM((_BSTRIDE,), jnp.int32),
          pltpu.VMEM((_BSTRIDE,), jnp.float32),
          pltpu.VMEM((_BSTRIDE,), jnp.float32),
          pltpu.VMEM((_BSTRIDE,), jnp.float32),
          pltpu.VMEM((704,), jnp.float32),
          pltpu.VMEM_SHARED((_TSIZE,), jnp.float32),
          pltpu.VMEM_SHARED((_DEGSZ,), jnp.float32),
          pltpu.VMEM_SHARED((_DEGSZ,), jnp.float32),
      ],
  )
  def count(src_hbm, dst_hbm, wv_hbm, w_out, dego_out, degi_out,
            codes, bpos, bcnt, wown, srcv, dstv,
            cidx, dsrc, ddst, dwv, cnts, ones, zeros,
            table, dego_sp, degi_sp):
    c = lax.axis_index("c")
    s = lax.axis_index("s")
    g = s * 2 + c
    iota = lax.iota(jnp.int32, 16)
    fzero = jnp.zeros((16,), jnp.float32)
    fone = jnp.ones((16,), jnp.float32)

    def zf(i, _):
      zeros[pl.ds(i * 16, 16)] = fzero
      return 0
    lax.fori_loop(0, 704 // 16, zf, 0)
    for k in range(_BSTRIDE // 16):
      ones[pl.ds(k * 16, 16)] = fone
    TPT = _TSIZE // 16
    def zt(q, _):
      pltpu.sync_copy(zeros, table.at[pl.ds(s * TPT + q * 704, 704)])
      return 0
    lax.fori_loop(0, TPT // 704, zt, 0)
    rem = TPT % 704
    pltpu.sync_copy(zeros.at[pl.ds(0, rem)],
                    table.at[pl.ds(s * TPT + (TPT // 704) * 704, rem)])
    DPT = _DEGSZ // 16
    pltpu.sync_copy(zeros.at[pl.ds(0, DPT)], dego_sp.at[pl.ds(s * DPT, DPT)])
    pltpu.sync_copy(zeros.at[pl.ds(0, DPT)], degi_sp.at[pl.ds(s * DPT, DPT)])
    pltpu.sync_copy(wv_hbm.at[pl.ds(g * _EPT, _EPT)],
                    wown.at[pl.ds(0, _EPT)])

    def cchunk(t, _):
      base = s * _EPS + t * _CH
      pltpu.sync_copy(src_hbm.at[pl.ds(base, _CH)], srcv)
      pltpu.sync_copy(dst_hbm.at[pl.ds(base, _CH)], dstv)
      for k in range(_CH // 16):
        sv = srcv[pl.ds(k * 16, 16)]
        dv = dstv[pl.ds(k * 16, 16)]
        codes[pl.ds(t * _CH + k * 16, 16)] = sv * _N + dv
      return 0
    lax.fori_loop(0, _EPS // _CH, cchunk, 0)

    def group(grp, _):
      for k in range(_GRP + 1):
        bcnt[pl.ds(k * 16, 16)] = jnp.zeros((16,), jnp.int32)

      @pl.loop(0, _EPS // 16, unroll=1)
      def bkt(j):
        cv = codes[pl.ds(j * 16, 16)]
        b = lax.shift_right_logical(cv, _TWS)
        bg = b - grp * _GRP
        bl = jnp.where((bg >= 0) & (bg < _GRP), bg, _GRP)
        ci = bl * 16 + iota
        cur = plsc.load_gather(bcnt, [ci])
        curc = jnp.minimum(cur, _BCAP - 1)
        addr = bl * _BSTRIDE + iota * _BCAP + curc
        plsc.store_scatter(bpos, [addr], j * 16 + iota)
        plsc.store_scatter(bcnt, [ci], cur + 1)
      plsc.subcore_barrier()

      def cpass(pp, _):
        kcnt = plsc.load_gather(bcnt, [pp * 16 + iota])
        lo = (grp * _GRP + pp) * _TW

        @pl.loop(0, _BCAP, unroll=1)
        def mk_lists(k):
          pos = plsc.load_gather(bpos, [pp * _BSTRIDE + iota * _BCAP + k])
          posc = jnp.clip(pos, 0, _EPS - 1)
          cv = plsc.load_gather(codes, [posc])
          live = kcnt > k
          dump = _TW + iota * _BCAP + k
          cidx[pl.ds(k * 16, 16)] = jnp.where(live, cv - lo, dump)
          own = (posc >= c * _EPT) & (posc < (c + 1) * _EPT)
          lived = live & own
          s0 = (cv.astype(jnp.float32) * (1.0 / _N)).astype(jnp.int32)
          r0 = cv - s0 * _N
          s1 = jnp.where(r0 < 0, s0 - 1, jnp.where(r0 >= _N, s0 + 1, s0))
          d1 = cv - s1 * _N
          ddump = _N + iota * _BCAP + k
          dsrc[pl.ds(k * 16, 16)] = jnp.where(lived, s1, ddump)
          ddst[pl.ds(k * 16, 16)] = jnp.where(lived, d1, ddump)
        pltpu.sync_copy(ones, table.at[cidx], add=True)
        plsc.subcore_barrier()
        pltpu.sync_copy(table.at[cidx], cnts)

        @pl.loop(0, _BCAP, unroll=1)
        def upd_w(k):
          pos = plsc.load_gather(bpos, [pp * _BSTRIDE + iota * _BCAP + k])
          posc = jnp.clip(pos, 0, _EPS - 1)
          live = kcnt > k
          own = (posc >= c * _EPT) & (posc < (c + 1) * _EPT)
          lived = live & own
          lown = jnp.where(lived, jnp.clip(posc - c * _EPT, 0, _EPT - 1),
                           _EPT + iota)
          wold = plsc.load_gather(wown, [lown])
          m = jnp.maximum(cnts[pl.ds(k * 16, 16)], 1.0)
          wnew = wold * (1.0 / m)
          plsc.store_scatter(wown, [lown], wnew)
          dwv[pl.ds(k * 16, 16)] = jnp.where(lived, wnew, 0.0)
        pltpu.sync_copy(dwv, dego_sp.at[dsrc], add=True)
        pltpu.sync_copy(dwv, degi_sp.at[ddst], add=True)
        plsc.subcore_barrier()
        pltpu.sync_copy(zeros.at[pl.ds(0, _BSTRIDE)], table.at[cidx])
        plsc.subcore_barrier()
        return 0
      lax.fori_loop(0, _GRP, cpass, 0)
      return 0
    lax.fori_loop(0, _NGRP, group, 0)

    pltpu.sync_copy(wown.at[pl.ds(0, _EPT)], w_out.at[pl.ds(g * _EPT, _EPT)])

    @pl.when(s == 0)
    def _():
      pltpu.sync_copy(dego_sp.at[pl.ds(0, _N)], dego_out.at[c])
      pltpu.sync_copy(degi_sp.at[pl.ds(0, _N)], degi_out.at[c])
  return count


def _make_agg(F):
  @functools.partial(
      pl.kernel, mesh=_mesh, compiler_params=_scparams,
      out_type=jax.ShapeDtypeStruct((2, _N, F), jnp.float32),
      scratch_types=[
          pltpu.VMEM((_CH,), jnp.int32),
          pltpu.VMEM((_CH,), jnp.int32),
          pltpu.VMEM((_CH,), jnp.float32),
          pltpu.VMEM((_CH, F), jnp.float32),
          pltpu.VMEM((_RPT, F), jnp.float32),
          pltpu.VMEM_SHARED((_N, F), jnp.float32),
          pltpu.SemaphoreType.DMA,
      ],
  )
  def agg(src_hbm, dst_hbm, w_hbm, m_hbm, out_hbm,
          sidx, didx, wch, rows, stage, table, sem):
    c = lax.axis_index("c")
    s = lax.axis_index("s")
    g = s * 2 + c

    zero = jnp.zeros((16,), jnp.float32)
    def zrow(r, _):
      for i in range(F // 16):
        stage[r, pl.ds(i * 16, 16)] = zero
      return 0
    lax.fori_loop(0, _RPT, zrow, 0)

    @pl.when(s < _N // _RPT)
    def _():
      pltpu.sync_copy(stage, table.at[pl.ds(s * _RPT, _RPT)])
    plsc.subcore_barrier()

    def chunk(t, _):
      base = g * _EPT + t * _CH
      pltpu.sync_copy(src_hbm.at[pl.ds(base, _CH)], sidx)
      pltpu.sync_copy(dst_hbm.at[pl.ds(base, _CH)], didx)
      pltpu.sync_copy(w_hbm.at[pl.ds(base, _CH)], wch)
      pltpu.async_copy(m_hbm.at[sidx], rows, sem).wait()
      for e0 in range(0, _CH, 16):
        wgrp = wch[pl.ds(e0, 16)]
        for j in range(16):
          e = e0 + j
          wv = jnp.broadcast_to(wgrp[j], (16,))
          for i in range(F // 16):
            rows[e, pl.ds(i * 16, 16)] = rows[e, pl.ds(i * 16, 16)] * wv
      pltpu.sync_copy(rows, table.at[didx], add=True)
      return 0
    lax.fori_loop(0, _NCH, chunk, 0)

    plsc.subcore_barrier()
    @pl.when(s < _N // _RPT)
    def _():
      pltpu.sync_copy(table.at[pl.ds(s * _RPT, _RPT)],
                      out_hbm.at[c, pl.ds(s * _RPT, _RPT)])
  return agg


_dots = _make_dots()
_count = _make_count()
_agg64 = _make_agg(_H)
_agg48 = _make_agg(_CP)


def _tc_norm_mm1(dego, degi, x, W1):
  B = 1000

  def body(dego_ref, degi_ref, x_ref, w_ref, m_ref, ns_ref, nd_ref):
    do = dego_ref[:, 0:1] + dego_ref[:, 1:2] + 1.0
    di = degi_ref[:, 0:1] + degi_ref[:, 1:2] + 1.0
    ns = lax.rsqrt(do)
    nd = lax.rsqrt(di)
    ns_ref[...] = ns
    nd_ref[...] = nd
    h = x_ref[...] * ns
    m_ref[...] = jnp.dot(h, w_ref[...], preferred_element_type=jnp.float32)

  return pl.pallas_call(
      body,
      grid=(_N // B,),
      in_specs=[
          pl.BlockSpec((B, 2), lambda i: (i, 0)),
          pl.BlockSpec((B, 2), lambda i: (i, 0)),
          pl.BlockSpec((B, _D), lambda i: (i, 0)),
          pl.BlockSpec((_D, _H), lambda i: (0, 0)),
      ],
      out_specs=[
          pl.BlockSpec((B, _H), lambda i: (i, 0)),
          pl.BlockSpec((B, 1), lambda i: (i, 0)),
          pl.BlockSpec((B, 1), lambda i: (i, 0)),
      ],
      out_shape=[
          jax.ShapeDtypeStruct((_N, _H), jnp.float32),
          jax.ShapeDtypeStruct((_N, 1), jnp.float32),
          jax.ShapeDtypeStruct((_N, 1), jnp.float32),
      ],
  )(dego, degi, x, W1)


def _tc_layer1_mm2(aggp, m1, ns, nd, b1, W2p):
  def body(a_ref, m_ref, ns_ref, nd_ref, b_ref, w_ref, m2_ref):
    agg = a_ref[0] + a_ref[1] + m_ref[...]
    o1 = jnp.maximum(agg * nd_ref[...] + b_ref[...], 0.0)
    h = o1 * ns_ref[...]
    m2_ref[...] = jnp.dot(h, w_ref[...], preferred_element_type=jnp.float32)

  B = 1000
  return pl.pallas_call(
      body,
      grid=(_N // B,),
      in_specs=[
          pl.BlockSpec((2, B, _H), lambda i: (0, i, 0)),
          pl.BlockSpec((B, _H), lambda i: (i, 0)),
          pl.BlockSpec((B, 1), lambda i: (i, 0)),
          pl.BlockSpec((B, 1), lambda i: (i, 0)),
          pl.BlockSpec((1, _H), lambda i: (0, 0)),
          pl.BlockSpec((_H, _CP), lambda i: (0, 0)),
      ],
      out_specs=pl.BlockSpec((B, _CP), lambda i: (i, 0)),
      out_shape=jax.ShapeDtypeStruct((_N, _CP), jnp.float32),
  )(aggp, m1, ns, nd, b1, W2p)


def _tc_final(aggp, m2, nd, b2):
  def body(a_ref, m_ref, nd_ref, b_ref, o_ref):
    agg = a_ref[0] + a_ref[1] + m_ref[...]
    o_ref[...] = agg[:, :_C] * nd_ref[...] + b_ref[...]

  B = 1000
  return pl.pallas_call(
      body,
      grid=(_N // B,),
      in_specs=[
          pl.BlockSpec((2, B, _CP), lambda i: (0, i, 0)),
          pl.BlockSpec((B, _CP), lambda i: (i, 0)),
          pl.BlockSpec((B, 1), lambda i: (i, 0)),
          pl.BlockSpec((1, _C), lambda i: (0, 0)),
      ],
      out_specs=pl.BlockSpec((B, _C), lambda i: (i, 0)),
      out_shape=jax.ShapeDtypeStruct((_N, _C), jnp.float32),
  )(aggp, m2, nd, b2)


def kernel(edge_index, features, W1, b1, W2, b2):
  src = edge_index[0].astype(jnp.int32)
  dst = edge_index[1].astype(jnp.int32)

  wv = _dots(src, dst, features)
  w, dego, degi = _count(src, dst, wv)
  m1, ns, nd = _tc_norm_mm1(dego.T, degi.T, features, W1)
  agg1 = _agg64(src, dst, w, m1)
  W2p = jnp.pad(W2, ((0, 0), (0, _CP - _C)))
  m2 = _tc_layer1_mm2(agg1, m1, ns, nd, b1[None, :], W2p)
  agg2 = _agg48(src, dst, w, m2)
  return _tc_final(agg2, m2, nd, b2[None, :])


# speedup vs baseline: 4.7270x; 1.4298x over previous
import functools
import jax
import jax.numpy as jnp
from jax import lax
from jax.experimental import pallas as pl
from jax.experimental.pallas import tpu as pltpu
from jax.experimental.pallas import tpu_sc as plsc

_N = 10000
_E = 320000
_D = 128
_H = 64
_C = 40
_CP = 48
_TILES = 32
_EPT = _E // _TILES
_CH = 80
_NCH = _EPT // _CH
_RPT = 1000

_EPS = _E // 16
_TWS = 20
_TW = 1 << _TWS
_NB = 96
_GRP = 12
_NGRP = _NB // _GRP
_BCAP = 32
_BSTRIDE = 16 * _BCAP
_TSIZE = _TW + 512
_DEGSZ = 11264

_mesh = plsc.VectorSubcoreMesh(core_axis_name="c", subcore_axis_name="s")
_scparams = pltpu.CompilerParams(needs_layout_passes=False,
                                 use_tc_tiling_on_sc=False)


def _make_dots():
  @functools.partial(
      pl.kernel, mesh=_mesh, compiler_params=_scparams,
      out_type=jax.ShapeDtypeStruct((_E,), jnp.float32),
      scratch_types=[
          pltpu.VMEM((_EPT,), jnp.float32),
          pltpu.VMEM((_CH, _D), jnp.float32),
          pltpu.VMEM((_CH, _D), jnp.float32),
          pltpu.VMEM((_CH, _D), jnp.float32),
          pltpu.VMEM((_CH, _D), jnp.float32),
          pltpu.VMEM((2, _CH), jnp.int32),
          pltpu.VMEM((2, _CH), jnp.int32),
          pltpu.VMEM((256,), jnp.float32),
          pltpu.SemaphoreType.DMA,
          pltpu.SemaphoreType.DMA,
          pltpu.SemaphoreType.DMA,
          pltpu.SemaphoreType.DMA,
          pltpu.SemaphoreType.DMA,
          pltpu.SemaphoreType.DMA,
      ],
  )
  def dots(ei_hbm, feat_hbm, w_out,
           wown, Xs0, Xs1, Xt0, Xt1, eib0, eib1, tsc,
           semei0, semei1, semxs0, semxs1, semxt0, semxt1):
    c = lax.axis_index("c")
    s = lax.axis_index("s")
    g = s * 2 + c
    iota = lax.iota(jnp.int32, 16)
    Xs = [Xs0, Xs1]
    Xt = [Xt0, Xt1]
    eib = [eib0, eib1]
    semei = [semei0, semei1]
    semxs = [semxs0, semxs1]
    semxt = [semxt0, semxt1]

    def start_ei(t, b):
      base = g * _EPT + t * _CH
      pltpu.async_copy(ei_hbm.at[:, pl.ds(base, _CH)], eib[b], semei[b])

    def wait_ei(b):
      pltpu.make_async_copy(ei_hbm.at[:, pl.ds(0, _CH)], eib[b],
                            semei[b]).wait()

    def start_g(b):
      pltpu.async_copy(feat_hbm.at[eib[b].at[0]], Xs[b], semxs[b])
      pltpu.async_copy(feat_hbm.at[eib[b].at[1]], Xt[b], semxt[b])

    def wait_g(b):
      pltpu.make_async_copy(feat_hbm.at[eib[b].at[0]], Xs[b],
                            semxs[b]).wait()
      pltpu.make_async_copy(feat_hbm.at[eib[b].at[1]], Xt[b],
                            semxt[b]).wait()

    start_ei(0, 0)
    start_ei(1, 1)
    wait_ei(0)
    start_g(0)

    @pl.loop(0, _NCH, step=2, unroll=1)
    def it(tb):
      for b in range(2):
        t = tb + b

        @pl.when(t < _NCH)
        def _():
          wait_g(b)

          @pl.when(t + 1 < _NCH)
          def _():
            wait_ei(1 - b)
            start_g(1 - b)

          svs = [eib[b][0, pl.ds(grp * 16, 16)] for grp in range(_CH // 16)]
          dvs = [eib[b][1, pl.ds(grp * 16, 16)] for grp in range(_CH // 16)]

          @pl.when(t + 2 < _NCH)
          def _():
            start_ei(t + 2, b)

          for grp in range(_CH // 16):
            for j in range(16):
              e = grp * 16 + j
              acc = Xs[b][e, pl.ds(0, 16)] * Xt[b][e, pl.ds(0, 16)]
              for i in range(1, _D // 16):
                acc = acc + (Xs[b][e, pl.ds(i * 16, 16)] *
                             Xt[b][e, pl.ds(i * 16, 16)])
              tsc[pl.ds(j * 16, 16)] = acc
            dotv = plsc.load_gather(tsc, [iota * 16])
            for col in range(1, 16):
              dotv = dotv + plsc.load_gather(tsc, [iota * 16 + col])
            keep = (dotv >= 0.0) & (svs[grp] != dvs[grp])
            wown[pl.ds(t * _CH + grp * 16, 16)] = jnp.where(keep, 1.0, 0.0)

    pltpu.sync_copy(wown, w_out.at[pl.ds(g * _EPT, _EPT)])
  return dots


def _make_count():
  @functools.partial(
      pl.kernel, mesh=_mesh, compiler_params=_scparams,
      out_type=[
          jax.ShapeDtypeStruct((_E,), jnp.float32),
          jax.ShapeDtypeStruct((2, _N), jnp.float32),
          jax.ShapeDtypeStruct((2, _N), jnp.float32),
      ],
      scratch_types=[
          pltpu.VMEM((_EPS,), jnp.int32),
          pltpu.VMEM(((_GRP + 1) * _BSTRIDE,), jnp.int32),
          pltpu.VMEM(((_GRP + 1) * 16,), jnp.int32),
          pltpu.VMEM((_EPT + 16,), jnp.float32),
          pltpu.VMEM((_CH,), jnp.int32),
          pltpu.VMEM((_CH,), jnp.int32),
          pltpu.VMEM((_BSTRIDE,), jnp.int32),
          pltpu.VMEM((_BSTRIDE,), jnp.int32),
          pltpu.VMEM((_BSTRIDE,), jnp.int32),
          pltpu.VMEM((_BSTRIDE,), jnp.float32),
          pltpu.VMEM((_BSTRIDE,), jnp.float32),
          pltpu.VMEM((_BSTRIDE,), jnp.float32),
          pltpu.VMEM((704,), jnp.float32),
          pltpu.VMEM_SHARED((_TSIZE,), jnp.float32),
          pltpu.VMEM_SHARED((_DEGSZ,), jnp.float32),
          pltpu.VMEM_SHARED((_DEGSZ,), jnp.float32),
      ],
  )
  def count(src_hbm, dst_hbm, wv_hbm, w_out, dego_out, degi_out,
            codes, bpos, bcnt, wown, srcv, dstv,
            cidx, dsrc, ddst, dwv, cnts, ones, zeros,
            table, dego_sp, degi_sp):
    c = lax.axis_index("c")
    s = lax.axis_index("s")
    g = s * 2 + c
    iota = lax.iota(jnp.int32, 16)
    fzero = jnp.zeros((16,), jnp.float32)
    fone = jnp.ones((16,), jnp.float32)

    def zf(i, _):
      zeros[pl.ds(i * 16, 16)] = fzero
      return 0
    lax.fori_loop(0, 704 // 16, zf, 0)
    for k in range(_BSTRIDE // 16):
      ones[pl.ds(k * 16, 16)] = fone
    TPT = _TSIZE // 16
    def zt(q, _):
      pltpu.sync_copy(zeros, table.at[pl.ds(s * TPT + q * 704, 704)])
      return 0
    lax.fori_loop(0, TPT // 704, zt, 0)
    rem = TPT % 704
    pltpu.sync_copy(zeros.at[pl.ds(0, rem)],
                    table.at[pl.ds(s * TPT + (TPT // 704) * 704, rem)])
    DPT = _DEGSZ // 16
    pltpu.sync_copy(zeros.at[pl.ds(0, DPT)], dego_sp.at[pl.ds(s * DPT, DPT)])
    pltpu.sync_copy(zeros.at[pl.ds(0, DPT)], degi_sp.at[pl.ds(s * DPT, DPT)])
    pltpu.sync_copy(wv_hbm.at[pl.ds(g * _EPT, _EPT)],
                    wown.at[pl.ds(0, _EPT)])

    def cchunk(t, _):
      base = s * _EPS + t * _CH
      pltpu.sync_copy(src_hbm.at[pl.ds(base, _CH)], srcv)
      pltpu.sync_copy(dst_hbm.at[pl.ds(base, _CH)], dstv)
      for k in range(_CH // 16):
        sv = srcv[pl.ds(k * 16, 16)]
        dv = dstv[pl.ds(k * 16, 16)]
        codes[pl.ds(t * _CH + k * 16, 16)] = sv * _N + dv
      return 0
    lax.fori_loop(0, _EPS // _CH, cchunk, 0)

    def group(grp, _):
      for k in range(_GRP + 1):
        bcnt[pl.ds(k * 16, 16)] = jnp.zeros((16,), jnp.int32)

      @pl.loop(0, _EPS // 16, unroll=1)
      def bkt(j):
        cv = codes[pl.ds(j * 16, 16)]
        b = lax.shift_right_logical(cv, _TWS)
        bg = b - grp * _GRP
        bl = jnp.where((bg >= 0) & (bg < _GRP), bg, _GRP)
        ci = bl * 16 + iota
        cur = plsc.load_gather(bcnt, [ci])
        curc = jnp.minimum(cur, _BCAP - 1)
        addr = bl * _BSTRIDE + iota * _BCAP + curc
        plsc.store_scatter(bpos, [addr], j * 16 + iota)
        plsc.store_scatter(bcnt, [ci], cur + 1)
      plsc.subcore_barrier()

      def cpass(pp, _):
        kcnt = plsc.load_gather(bcnt, [pp * 16 + iota])
        lo = (grp * _GRP + pp) * _TW

        @pl.loop(0, _BCAP, unroll=1)
        def mk_lists(k):
          pos = plsc.load_gather(bpos, [pp * _BSTRIDE + iota * _BCAP + k])
          posc = jnp.clip(pos, 0, _EPS - 1)
          cv = plsc.load_gather(codes, [posc])
          live = kcnt > k
          dump = _TW + iota * _BCAP + k
          cidx[pl.ds(k * 16, 16)] = jnp.where(live, cv - lo, dump)
          own = (posc >= c * _EPT) & (posc < (c + 1) * _EPT)
          lived = live & own
          s0 = (cv.astype(jnp.float32) * (1.0 / _N)).astype(jnp.int32)
          r0 = cv - s0 * _N
          s1 = jnp.where(r0 < 0, s0 - 1, jnp.where(r0 >= _N, s0 + 1, s0))
          d1 = cv - s1 * _N
          ddump = _N + iota * _BCAP + k
          dsrc[pl.ds(k * 16, 16)] = jnp.where(lived, s1, ddump)
          ddst[pl.ds(k * 16, 16)] = jnp.where(lived, d1, ddump)
        pltpu.sync_copy(ones, table.at[cidx], add=True)
        plsc.subcore_barrier()
        pltpu.sync_copy(table.at[cidx], cnts)

        @pl.loop(0, _BCAP, unroll=1)
        def upd_w(k):
          pos = plsc.load_gather(bpos, [pp * _BSTRIDE + iota * _BCAP + k])
          posc = jnp.clip(pos, 0, _EPS - 1)
          live = kcnt > k
          own = (posc >= c * _EPT) & (posc < (c + 1) * _EPT)
          lived = live & own
          lown = jnp.where(lived, jnp.clip(posc - c * _EPT, 0, _EPT - 1),
                           _EPT + iota)
          wold = plsc.load_gather(wown, [lown])
          m = jnp.maximum(cnts[pl.ds(k * 16, 16)], 1.0)
          wnew = wold * (1.0 / m)
          plsc.store_scatter(wown, [lown], wnew)
          dwv[pl.ds(k * 16, 16)] = jnp.where(lived, wnew, 0.0)
        pltpu.sync_copy(dwv, dego_sp.at[dsrc], add=True)
        pltpu.sync_copy(dwv, degi_sp.at[ddst], add=True)
        plsc.subcore_barrier()
        pltpu.sync_copy(zeros.at[pl.ds(0, _BSTRIDE)], table.at[cidx])
        plsc.subcore_barrier()
        return 0
      lax.fori_loop(0, _GRP, cpass, 0)
      return 0
    lax.fori_loop(0, _NGRP, group, 0)

    pltpu.sync_copy(wown.at[pl.ds(0, _EPT)], w_out.at[pl.ds(g * _EPT, _EPT)])

    @pl.when(s == 0)
    def _():
      pltpu.sync_copy(dego_sp.at[pl.ds(0, _N)], dego_out.at[c])
      pltpu.sync_copy(degi_sp.at[pl.ds(0, _N)], degi_out.at[c])
  return count


def _make_agg(F):
  @functools.partial(
      pl.kernel, mesh=_mesh, compiler_params=_scparams,
      out_type=jax.ShapeDtypeStruct((2, _N, F), jnp.float32),
      scratch_types=[
          pltpu.VMEM((2, _CH), jnp.int32),
          pltpu.VMEM((2, _CH), jnp.int32),
          pltpu.VMEM((_CH,), jnp.float32),
          pltpu.VMEM((_CH,), jnp.float32),
          pltpu.VMEM((_CH, F), jnp.float32),
          pltpu.VMEM((_CH, F), jnp.float32),
          pltpu.VMEM((_RPT, F), jnp.float32),
          pltpu.VMEM_SHARED((_N, F), jnp.float32),
          pltpu.SemaphoreType.DMA,
          pltpu.SemaphoreType.DMA,
          pltpu.SemaphoreType.DMA,
          pltpu.SemaphoreType.DMA,
          pltpu.SemaphoreType.DMA,
          pltpu.SemaphoreType.DMA,
      ],
  )
  def agg(ei_hbm, w_hbm, m_hbm, out_hbm,
          eib0, eib1, wch0, wch1, rows0, rows1, stage, table,
          semei0, semei1, semw0, semw1, semg0, semg1):
    c = lax.axis_index("c")
    s = lax.axis_index("s")
    g = s * 2 + c
    eib = [eib0, eib1]
    wch = [wch0, wch1]
    rows = [rows0, rows1]
    semei = [semei0, semei1]
    semw = [semw0, semw1]
    semg = [semg0, semg1]

    zero = jnp.zeros((16,), jnp.float32)
    def zrow(r, _):
      for i in range(F // 16):
        stage[r, pl.ds(i * 16, 16)] = zero
      return 0
    lax.fori_loop(0, _RPT, zrow, 0)

    @pl.when(s < _N // _RPT)
    def _():
      pltpu.sync_copy(stage, table.at[pl.ds(s * _RPT, _RPT)])

    def start_eiw(t, b):
      base = g * _EPT + t * _CH
      pltpu.async_copy(ei_hbm.at[:, pl.ds(base, _CH)], eib[b], semei[b])
      pltpu.async_copy(w_hbm.at[pl.ds(base, _CH)], wch[b], semw[b])

    def wait_ei(b):
      pltpu.make_async_copy(ei_hbm.at[:, pl.ds(0, _CH)], eib[b],
                            semei[b]).wait()

    def wait_w(b):
      pltpu.make_async_copy(w_hbm.at[pl.ds(0, _CH)], wch[b], semw[b]).wait()

    def start_g(b):
      pltpu.async_copy(m_hbm.at[eib[b].at[0]], rows[b], semg[b])

    def wait_g(b):
      pltpu.make_async_copy(m_hbm.at[eib[b].at[0]], rows[b], semg[b]).wait()

    start_eiw(0, 0)
    start_eiw(1, 1)
    plsc.subcore_barrier()
    wait_ei(0)
    start_g(0)

    @pl.loop(0, _NCH, step=2, unroll=1)
    def it(tb):
      for b in range(2):
        t = tb + b

        @pl.when(t < _NCH)
        def _():
          wait_g(b)
          wait_w(b)

          @pl.when(t + 1 < _NCH)
          def _():
            wait_ei(1 - b)
            start_g(1 - b)

          for e0 in range(0, _CH, 16):
            wgrp = wch[b][pl.ds(e0, 16)]
            for j in range(16):
              e = e0 + j
              wv = jnp.broadcast_to(wgrp[j], (16,))
              for i in range(F // 16):
                rows[b][e, pl.ds(i * 16, 16)] = (
                    rows[b][e, pl.ds(i * 16, 16)] * wv)
          pltpu.sync_copy(rows[b], table.at[eib[b].at[1]], add=True)

          @pl.when(t + 2 < _NCH)
          def _():
            start_eiw(t + 2, b)

    plsc.subcore_barrier()
    @pl.when(s < _N // _RPT)
    def _():
      pltpu.sync_copy(table.at[pl.ds(s * _RPT, _RPT)],
                      out_hbm.at[c, pl.ds(s * _RPT, _RPT)])
  return agg


_dots = _make_dots()
_count = _make_count()
_agg64 = _make_agg(_H)
_agg48 = _make_agg(_CP)


def _tc_norm_mm1(dego, degi, x, W1):
  B = 1000

  def body(dego_ref, degi_ref, x_ref, w_ref, m_ref, ns_ref, nd_ref):
    do = dego_ref[:, 0:1] + dego_ref[:, 1:2] + 1.0
    di = degi_ref[:, 0:1] + degi_ref[:, 1:2] + 1.0
    ns = lax.rsqrt(do)
    nd = lax.rsqrt(di)
    ns_ref[...] = ns
    nd_ref[...] = nd
    h = x_ref[...] * ns
    m_ref[...] = jnp.dot(h, w_ref[...], preferred_element_type=jnp.float32)

  return pl.pallas_call(
      body,
      grid=(_N // B,),
      in_specs=[
          pl.BlockSpec((B, 2), lambda i: (i, 0)),
          pl.BlockSpec((B, 2), lambda i: (i, 0)),
          pl.BlockSpec((B, _D), lambda i: (i, 0)),
          pl.BlockSpec((_D, _H), lambda i: (0, 0)),
      ],
      out_specs=[
          pl.BlockSpec((B, _H), lambda i: (i, 0)),
          pl.BlockSpec((B, 1), lambda i: (i, 0)),
          pl.BlockSpec((B, 1), lambda i: (i, 0)),
      ],
      out_shape=[
          jax.ShapeDtypeStruct((_N, _H), jnp.float32),
          jax.ShapeDtypeStruct((_N, 1), jnp.float32),
          jax.ShapeDtypeStruct((_N, 1), jnp.float32),
      ],
  )(dego, degi, x, W1)


def _tc_layer1_mm2(aggp, m1, ns, nd, b1, W2p):
  def body(a_ref, m_ref, ns_ref, nd_ref, b_ref, w_ref, m2_ref):
    agg = a_ref[0] + a_ref[1] + m_ref[...]
    o1 = jnp.maximum(agg * nd_ref[...] + b_ref[...], 0.0)
    h = o1 * ns_ref[...]
    m2_ref[...] = jnp.dot(h, w_ref[...], preferred_element_type=jnp.float32)

  B = 1000
  return pl.pallas_call(
      body,
      grid=(_N // B,),
      in_specs=[
          pl.BlockSpec((2, B, _H), lambda i: (0, i, 0)),
          pl.BlockSpec((B, _H), lambda i: (i, 0)),
          pl.BlockSpec((B, 1), lambda i: (i, 0)),
          pl.BlockSpec((B, 1), lambda i: (i, 0)),
          pl.BlockSpec((1, _H), lambda i: (0, 0)),
          pl.BlockSpec((_H, _CP), lambda i: (0, 0)),
      ],
      out_specs=pl.BlockSpec((B, _CP), lambda i: (i, 0)),
      out_shape=jax.ShapeDtypeStruct((_N, _CP), jnp.float32),
  )(aggp, m1, ns, nd, b1, W2p)


def _tc_final(aggp, m2, nd, b2):
  def body(a_ref, m_ref, nd_ref, b_ref, o_ref):
    agg = a_ref[0] + a_ref[1] + m_ref[...]
    o_ref[...] = agg[:, :_C] * nd_ref[...] + b_ref[...]

  B = 1000
  return pl.pallas_call(
      body,
      grid=(_N // B,),
      in_specs=[
          pl.BlockSpec((2, B, _CP), lambda i: (0, i, 0)),
          pl.BlockSpec((B, _CP), lambda i: (i, 0)),
          pl.BlockSpec((B, 1), lambda i: (i, 0)),
          pl.BlockSpec((1, _C), lambda i: (0, 0)),
      ],
      out_specs=pl.BlockSpec((B, _C), lambda i: (i, 0)),
      out_shape=jax.ShapeDtypeStruct((_N, _C), jnp.float32),
  )(aggp, m2, nd, b2)


def kernel(edge_index, features, W1, b1, W2, b2):
  ei = edge_index.astype(jnp.int32)
  src = ei[0]
  dst = ei[1]

  wv = _dots(ei, features)
  w, dego, degi = _count(src, dst, wv)
  m1, ns, nd = _tc_norm_mm1(dego.T, degi.T, features, W1)
  agg1 = _agg64(ei, w, m1)
  W2p = jnp.pad(W2, ((0, 0), (0, _CP - _C)))
  m2 = _tc_layer1_mm2(agg1, m1, ns, nd, b1[None, :], W2p)
  agg2 = _agg48(ei, w, m2)
  return _tc_final(agg2, m2, nd, b2[None, :])


# speedup vs baseline: 5.0045x; 1.0587x over previous
import functools
import jax
import jax.numpy as jnp
from jax import lax
from jax.experimental import pallas as pl
from jax.experimental.pallas import tpu as pltpu
from jax.experimental.pallas import tpu_sc as plsc

_N = 10000
_E = 320000
_D = 128
_H = 64
_C = 40
_CP = 48
_TILES = 32
_EPT = _E // _TILES
_CH = 80
_NCH = _EPT // _CH
_RPT = 1000

_EPS = _E // 16
_TWS = 21
_TW = 1 << _TWS
_TBL = 1 << 20
_NB = 48
_GRP = 12
_NGRP = _NB // _GRP
_BCAP = 64
_BSTRIDE = 16 * _BCAP
_TSIZE = _TBL + 1024
_DEGSZ = 12032

_mesh = plsc.VectorSubcoreMesh(core_axis_name="c", subcore_axis_name="s")
_scparams = pltpu.CompilerParams(needs_layout_passes=False,
                                 use_tc_tiling_on_sc=False)


def _make_dots():
  @functools.partial(
      pl.kernel, mesh=_mesh, compiler_params=_scparams,
      out_type=jax.ShapeDtypeStruct((_E,), jnp.float32),
      scratch_types=[
          pltpu.VMEM((_EPT,), jnp.float32),
          pltpu.VMEM((_CH, _D), jnp.float32),
          pltpu.VMEM((_CH, _D), jnp.float32),
          pltpu.VMEM((_CH, _D), jnp.float32),
          pltpu.VMEM((_CH, _D), jnp.float32),
          pltpu.VMEM((2, _CH), jnp.int32),
          pltpu.VMEM((2, _CH), jnp.int32),
          pltpu.VMEM((256,), jnp.float32),
          pltpu.SemaphoreType.DMA,
          pltpu.SemaphoreType.DMA,
          pltpu.SemaphoreType.DMA,
          pltpu.SemaphoreType.DMA,
          pltpu.SemaphoreType.DMA,
          pltpu.SemaphoreType.DMA,
      ],
  )
  def dots(ei_hbm, feat_hbm, w_out,
           wown, Xs0, Xs1, Xt0, Xt1, eib0, eib1, tsc,
           semei0, semei1, semxs0, semxs1, semxt0, semxt1):
    c = lax.axis_index("c")
    s = lax.axis_index("s")
    g = s * 2 + c
    iota = lax.iota(jnp.int32, 16)
    Xs = [Xs0, Xs1]
    Xt = [Xt0, Xt1]
    eib = [eib0, eib1]
    semei = [semei0, semei1]
    semxs = [semxs0, semxs1]
    semxt = [semxt0, semxt1]

    def start_ei(t, b):
      base = g * _EPT + t * _CH
      pltpu.async_copy(ei_hbm.at[:, pl.ds(base, _CH)], eib[b], semei[b])

    def wait_ei(b):
      pltpu.make_async_copy(ei_hbm.at[:, pl.ds(0, _CH)], eib[b],
                            semei[b]).wait()

    def start_g(b):
      pltpu.async_copy(feat_hbm.at[eib[b].at[0]], Xs[b], semxs[b])
      pltpu.async_copy(feat_hbm.at[eib[b].at[1]], Xt[b], semxt[b])

    def wait_g(b):
      pltpu.make_async_copy(feat_hbm.at[eib[b].at[0]], Xs[b],
                            semxs[b]).wait()
      pltpu.make_async_copy(feat_hbm.at[eib[b].at[1]], Xt[b],
                            semxt[b]).wait()

    start_ei(0, 0)
    start_ei(1, 1)
    wait_ei(0)
    start_g(0)

    @pl.loop(0, _NCH, step=2, unroll=1)
    def it(tb):
      for b in range(2):
        t = tb + b

        @pl.when(t < _NCH)
        def _():
          wait_g(b)

          @pl.when(t + 1 < _NCH)
          def _():
            wait_ei(1 - b)
            start_g(1 - b)

          svs = [eib[b][0, pl.ds(grp * 16, 16)] for grp in range(_CH // 16)]
          dvs = [eib[b][1, pl.ds(grp * 16, 16)] for grp in range(_CH // 16)]

          @pl.when(t + 2 < _NCH)
          def _():
            start_ei(t + 2, b)

          for grp in range(_CH // 16):
            for j in range(16):
              e = grp * 16 + j
              acc = Xs[b][e, pl.ds(0, 16)] * Xt[b][e, pl.ds(0, 16)]
              for i in range(1, _D // 16):
                acc = acc + (Xs[b][e, pl.ds(i * 16, 16)] *
                             Xt[b][e, pl.ds(i * 16, 16)])
              tsc[pl.ds(j * 16, 16)] = acc
            dotv = plsc.load_gather(tsc, [iota * 16])
            for col in range(1, 16):
              dotv = dotv + plsc.load_gather(tsc, [iota * 16 + col])
            keep = (dotv >= 0.0) & (svs[grp] != dvs[grp])
            wown[pl.ds(t * _CH + grp * 16, 16)] = jnp.where(keep, 1.0, 0.0)

    pltpu.sync_copy(wown, w_out.at[pl.ds(g * _EPT, _EPT)])
  return dots


def _make_count():
  @functools.partial(
      pl.kernel, mesh=_mesh, compiler_params=_scparams,
      out_type=[
          jax.ShapeDtypeStruct((_E,), jnp.float32),
          jax.ShapeDtypeStruct((2, _N), jnp.float32),
          jax.ShapeDtypeStruct((2, _N), jnp.float32),
      ],
      scratch_types=[
          pltpu.VMEM((_EPS,), jnp.int32),
          pltpu.VMEM(((_GRP + 1) * _BSTRIDE,), jnp.int32),
          pltpu.VMEM(((_GRP + 1) * 16,), jnp.int32),
          pltpu.VMEM((_EPT + 16,), jnp.float32),
          pltpu.VMEM((_CH,), jnp.int32),
          pltpu.VMEM((_CH,), jnp.int32),
          pltpu.VMEM((_BSTRIDE,), jnp.int32),
          pltpu.VMEM((_BSTRIDE,), jnp.int32),
          pltpu.VMEM((_BSTRIDE,), jnp.int32),
          pltpu.VMEM((_BSTRIDE,), jnp.float32),
          pltpu.VMEM((_BSTRIDE,), jnp.float32),
          pltpu.VMEM((_BSTRIDE,), jnp.float32),
          pltpu.VMEM((1024,), jnp.float32),
          pltpu.VMEM_SHARED((_TSIZE,), jnp.float32),
          pltpu.VMEM_SHARED((_DEGSZ,), jnp.float32),
          pltpu.VMEM_SHARED((_DEGSZ,), jnp.float32),
      ],
  )
  def count(src_hbm, dst_hbm, wv_hbm, w_out, dego_out, degi_out,
            codes, bpos, bcnt, wown, srcv, dstv,
            cidx, dsrc, ddst, dwv, cnts, addv, zeros,
            table, dego_sp, degi_sp):
    c = lax.axis_index("c")
    s = lax.axis_index("s")
    g = s * 2 + c
    iota = lax.iota(jnp.int32, 16)
    fzero = jnp.zeros((16,), jnp.float32)
    fone = jnp.ones((16,), jnp.float32)

    def zf(i, _):
      zeros[pl.ds(i * 16, 16)] = fzero
      return 0
    lax.fori_loop(0, 1024 // 16, zf, 0)
    TPT = _TSIZE // 16
    def zt(q, _):
      pltpu.sync_copy(zeros, table.at[pl.ds(s * TPT + q * 1024, 1024)])
      return 0
    lax.fori_loop(0, TPT // 1024, zt, 0)
    rem = TPT % 1024
    pltpu.sync_copy(zeros.at[pl.ds(0, rem)],
                    table.at[pl.ds(s * TPT + (TPT // 1024) * 1024, rem)])
    DPT = _DEGSZ // 16
    pltpu.sync_copy(zeros.at[pl.ds(0, DPT)], dego_sp.at[pl.ds(s * DPT, DPT)])
    pltpu.sync_copy(zeros.at[pl.ds(0, DPT)], degi_sp.at[pl.ds(s * DPT, DPT)])
    pltpu.sync_copy(wv_hbm.at[pl.ds(g * _EPT, _EPT)],
                    wown.at[pl.ds(0, _EPT)])

    def cchunk(t, _):
      base = s * _EPS + t * _CH
      pltpu.sync_copy(src_hbm.at[pl.ds(base, _CH)], srcv)
      pltpu.sync_copy(dst_hbm.at[pl.ds(base, _CH)], dstv)
      for k in range(_CH // 16):
        sv = srcv[pl.ds(k * 16, 16)]
        dv = dstv[pl.ds(k * 16, 16)]
        codes[pl.ds(t * _CH + k * 16, 16)] = sv * _N + dv
      return 0
    lax.fori_loop(0, _EPS // _CH, cchunk, 0)

    def group(grp, _):
      for k in range(_GRP + 1):
        bcnt[pl.ds(k * 16, 16)] = jnp.zeros((16,), jnp.int32)

      @pl.loop(0, _EPS // 16, unroll=1)
      def bkt(j):
        cv = codes[pl.ds(j * 16, 16)]
        b = lax.shift_right_logical(cv, _TWS)
        bg = b - grp * _GRP
        bl = jnp.where((bg >= 0) & (bg < _GRP), bg, _GRP)
        ci = bl * 16 + iota
        cur = plsc.load_gather(bcnt, [ci])
        curc = jnp.minimum(cur, _BCAP - 1)
        addr = bl * _BSTRIDE + iota * _BCAP + curc
        plsc.store_scatter(bpos, [addr], j * 16 + iota)
        plsc.store_scatter(bcnt, [ci], cur + 1)
      plsc.subcore_barrier()

      def cpass(pp, _):
        kcnt = plsc.load_gather(bcnt, [pp * 16 + iota])
        lo = (grp * _GRP + pp) * _TW

        @pl.loop(0, _BCAP, unroll=1)
        def mk_lists(k):
          pos = plsc.load_gather(bpos, [pp * _BSTRIDE + iota * _BCAP + k])
          posc = jnp.clip(pos, 0, _EPS - 1)
          cv = plsc.load_gather(codes, [posc])
          live = kcnt > k
          dump = _TBL + iota * _BCAP + k
          cidx[pl.ds(k * 16, 16)] = jnp.where(
              live, lax.shift_right_logical(cv - lo, 1), dump)
          addv[pl.ds(k * 16, 16)] = jnp.where(
              (cv & 1) == 0, 1.0, 65536.0)
          own = (posc >= c * _EPT) & (posc < (c + 1) * _EPT)
          lived = live & own
          s0 = (cv.astype(jnp.float32) * (1.0 / _N)).astype(jnp.int32)
          r0 = cv - s0 * _N
          s1 = jnp.where(r0 < 0, s0 - 1, jnp.where(r0 >= _N, s0 + 1, s0))
          d1 = cv - s1 * _N
          ddump = _N + iota * _BCAP + k
          dsrc[pl.ds(k * 16, 16)] = jnp.where(lived, s1, ddump)
          ddst[pl.ds(k * 16, 16)] = jnp.where(lived, d1, ddump)
        pltpu.sync_copy(addv, table.at[cidx], add=True)
        plsc.subcore_barrier()
        pltpu.sync_copy(table.at[cidx], cnts)

        @pl.loop(0, _BCAP, unroll=1)
        def upd_w(k):
          pos = plsc.load_gather(bpos, [pp * _BSTRIDE + iota * _BCAP + k])
          posc = jnp.clip(pos, 0, _EPS - 1)
          cv = plsc.load_gather(codes, [posc])
          live = kcnt > k
          own = (posc >= c * _EPT) & (posc < (c + 1) * _EPT)
          lived = live & own
          lown = jnp.where(lived, jnp.clip(posc - c * _EPT, 0, _EPT - 1),
                           _EPT + iota)
          wold = plsc.load_gather(wown, [lown])
          v = cnts[pl.ds(k * 16, 16)]
          hi = (v * (1.0 / 65536.0)).astype(jnp.int32).astype(jnp.float32)
          lo16 = v - hi * 65536.0
          m = jnp.maximum(jnp.where((cv & 1) == 0, lo16, hi), 1.0)
          wnew = wold * (1.0 / m)
          plsc.store_scatter(wown, [lown], wnew)
          dwv[pl.ds(k * 16, 16)] = jnp.where(lived, wnew, 0.0)
        pltpu.sync_copy(dwv, dego_sp.at[dsrc], add=True)
        pltpu.sync_copy(dwv, degi_sp.at[ddst], add=True)
        plsc.subcore_barrier()
        pltpu.sync_copy(zeros.at[pl.ds(0, _BSTRIDE)], table.at[cidx])
        plsc.subcore_barrier()
        return 0
      lax.fori_loop(0, _GRP, cpass, 0)
      return 0
    lax.fori_loop(0, _NGRP, group, 0)

    pltpu.sync_copy(wown.at[pl.ds(0, _EPT)], w_out.at[pl.ds(g * _EPT, _EPT)])

    @pl.when(s == 0)
    def _():
      pltpu.sync_copy(dego_sp.at[pl.ds(0, _N)], dego_out.at[c])
      pltpu.sync_copy(degi_sp.at[pl.ds(0, _N)], degi_out.at[c])
  return count


def _make_agg(F):
  @functools.partial(
      pl.kernel, mesh=_mesh, compiler_params=_scparams,
      out_type=jax.ShapeDtypeStruct((2, _N, F), jnp.float32),
      scratch_types=[
          pltpu.VMEM((2, _CH), jnp.int32),
          pltpu.VMEM((2, _CH), jnp.int32),
          pltpu.VMEM((_CH,), jnp.float32),
          pltpu.VMEM((_CH,), jnp.float32),
          pltpu.VMEM((_CH, F), jnp.float32),
          pltpu.VMEM((_CH, F), jnp.float32),
          pltpu.VMEM((_RPT, F), jnp.float32),
          pltpu.VMEM_SHARED((_N, F), jnp.float32),
          pltpu.SemaphoreType.DMA,
          pltpu.SemaphoreType.DMA,
          pltpu.SemaphoreType.DMA,
          pltpu.SemaphoreType.DMA,
          pltpu.SemaphoreType.DMA,
          pltpu.SemaphoreType.DMA,
      ],
  )
  def agg(ei_hbm, w_hbm, m_hbm, out_hbm,
          eib0, eib1, wch0, wch1, rows0, rows1, stage, table,
          semei0, semei1, semw0, semw1, semg0, semg1):
    c = lax.axis_index("c")
    s = lax.axis_index("s")
    g = s * 2 + c
    eib = [eib0, eib1]
    wch = [wch0, wch1]
    rows = [rows0, rows1]
    semei = [semei0, semei1]
    semw = [semw0, semw1]
    semg = [semg0, semg1]

    zero = jnp.zeros((16,), jnp.float32)
    def zrow(r, _):
      for i in range(F // 16):
        stage[r, pl.ds(i * 16, 16)] = zero
      return 0
    lax.fori_loop(0, _RPT, zrow, 0)

    @pl.when(s < _N // _RPT)
    def _():
      pltpu.sync_copy(stage, table.at[pl.ds(s * _RPT, _RPT)])

    def start_eiw(t, b):
      base = g * _EPT + t * _CH
      pltpu.async_copy(ei_hbm.at[:, pl.ds(base, _CH)], eib[b], semei[b])
      pltpu.async_copy(w_hbm.at[pl.ds(base, _CH)], wch[b], semw[b])

    def wait_ei(b):
      pltpu.make_async_copy(ei_hbm.at[:, pl.ds(0, _CH)], eib[b],
                            semei[b]).wait()

    def wait_w(b):
      pltpu.make_async_copy(w_hbm.at[pl.ds(0, _CH)], wch[b], semw[b]).wait()

    def start_g(b):
      pltpu.async_copy(m_hbm.at[eib[b].at[0]], rows[b], semg[b])

    def wait_g(b):
      pltpu.make_async_copy(m_hbm.at[eib[b].at[0]], rows[b], semg[b]).wait()

    start_eiw(0, 0)
    start_eiw(1, 1)
    plsc.subcore_barrier()
    wait_ei(0)
    start_g(0)

    @pl.loop(0, _NCH, step=2, unroll=1)
    def it(tb):
      for b in range(2):
        t = tb + b

        @pl.when(t < _NCH)
        def _():
          wait_g(b)
          wait_w(b)

          @pl.when(t + 1 < _NCH)
          def _():
            wait_ei(1 - b)
            start_g(1 - b)

          for e0 in range(0, _CH, 16):
            wgrp = wch[b][pl.ds(e0, 16)]
            for j in range(16):
              e = e0 + j
              wv = jnp.broadcast_to(wgrp[j], (16,))
              for i in range(F // 16):
                rows[b][e, pl.ds(i * 16, 16)] = (
                    rows[b][e, pl.ds(i * 16, 16)] * wv)
          pltpu.sync_copy(rows[b], table.at[eib[b].at[1]], add=True)

          @pl.when(t + 2 < _NCH)
          def _():
            start_eiw(t + 2, b)

    plsc.subcore_barrier()
    @pl.when(s < _N // _RPT)
    def _():
      pltpu.sync_copy(table.at[pl.ds(s * _RPT, _RPT)],
                      out_hbm.at[c, pl.ds(s * _RPT, _RPT)])
  return agg


_dots = _make_dots()
_count = _make_count()
_agg64 = _make_agg(_H)
_agg48 = _make_agg(_CP)


def _tc_norm_mm1(dego, degi, x, W1):
  B = 1000

  def body(dego_ref, degi_ref, x_ref, w_ref, m_ref, ns_ref, nd_ref):
    do = dego_ref[:, 0:1] + dego_ref[:, 1:2] + 1.0
    di = degi_ref[:, 0:1] + degi_ref[:, 1:2] + 1.0
    ns = lax.rsqrt(do)
    nd = lax.rsqrt(di)
    ns_ref[...] = ns
    nd_ref[...] = nd
    h = x_ref[...] * ns
    m_ref[...] = jnp.dot(h, w_ref[...], preferred_element_type=jnp.float32)

  return pl.pallas_call(
      body,
      grid=(_N // B,),
      in_specs=[
          pl.BlockSpec((B, 2), lambda i: (i, 0)),
          pl.BlockSpec((B, 2), lambda i: (i, 0)),
          pl.BlockSpec((B, _D), lambda i: (i, 0)),
          pl.BlockSpec((_D, _H), lambda i: (0, 0)),
      ],
      out_specs=[
          pl.BlockSpec((B, _H), lambda i: (i, 0)),
          pl.BlockSpec((B, 1), lambda i: (i, 0)),
          pl.BlockSpec((B, 1), lambda i: (i, 0)),
      ],
      out_shape=[
          jax.ShapeDtypeStruct((_N, _H), jnp.float32),
          jax.ShapeDtypeStruct((_N, 1), jnp.float32),
          jax.ShapeDtypeStruct((_N, 1), jnp.float32),
      ],
  )(dego, degi, x, W1)


def _tc_layer1_mm2(aggp, m1, ns, nd, b1, W2p):
  def body(a_ref, m_ref, ns_ref, nd_ref, b_ref, w_ref, m2_ref):
    agg = a_ref[0] + a_ref[1] + m_ref[...]
    o1 = jnp.maximum(agg * nd_ref[...] + b_ref[...], 0.0)
    h = o1 * ns_ref[...]
    m2_ref[...] = jnp.dot(h, w_ref[...], preferred_element_type=jnp.float32)

  B = 1000
  return pl.pallas_call(
      body,
      grid=(_N // B,),
      in_specs=[
          pl.BlockSpec((2, B, _H), lambda i: (0, i, 0)),
          pl.BlockSpec((B, _H), lambda i: (i, 0)),
          pl.BlockSpec((B, 1), lambda i: (i, 0)),
          pl.BlockSpec((B, 1), lambda i: (i, 0)),
          pl.BlockSpec((1, _H), lambda i: (0, 0)),
          pl.BlockSpec((_H, _CP), lambda i: (0, 0)),
      ],
      out_specs=pl.BlockSpec((B, _CP), lambda i: (i, 0)),
      out_shape=jax.ShapeDtypeStruct((_N, _CP), jnp.float32),
  )(aggp, m1, ns, nd, b1, W2p)


def _tc_final(aggp, m2, nd, b2):
  def body(a_ref, m_ref, nd_ref, b_ref, o_ref):
    agg = a_ref[0] + a_ref[1] + m_ref[...]
    o_ref[...] = agg[:, :_C] * nd_ref[...] + b_ref[...]

  B = 1000
  return pl.pallas_call(
      body,
      grid=(_N // B,),
      in_specs=[
          pl.BlockSpec((2, B, _CP), lambda i: (0, i, 0)),
          pl.BlockSpec((B, _CP), lambda i: (i, 0)),
          pl.BlockSpec((B, 1), lambda i: (i, 0)),
          pl.BlockSpec((1, _C), lambda i: (0, 0)),
      ],
      out_specs=pl.BlockSpec((B, _C), lambda i: (i, 0)),
      out_shape=jax.ShapeDtypeStruct((_N, _C), jnp.float32),
  )(aggp, m2, nd, b2)


def kernel(edge_index, features, W1, b1, W2, b2):
  ei = edge_index.astype(jnp.int32)
  src = ei[0]
  dst = ei[1]

  wv = _dots(ei, features)
  w, dego, degi = _count(src, dst, wv)
  m1, ns, nd = _tc_norm_mm1(dego.T, degi.T, features, W1)
  agg1 = _agg64(ei, w, m1)
  W2p = jnp.pad(W2, ((0, 0), (0, _CP - _C)))
  m2 = _tc_layer1_mm2(agg1, m1, ns, nd, b1[None, :], W2p)
  agg2 = _agg48(ei, w, m2)
  return _tc_final(agg2, m2, nd, b2[None, :])


# speedup vs baseline: 5.8096x; 1.1609x over previous
import functools
import jax
import jax.numpy as jnp
from jax import lax
from jax.experimental import pallas as pl
from jax.experimental.pallas import tpu as pltpu
from jax.experimental.pallas import tpu_sc as plsc

_N = 10000
_E = 320000
_D = 128
_H = 64
_C = 40
_CP = 48
_TILES = 32
_EPT = _E // _TILES
_CH = 80
_NCH = _EPT // _CH
_RPT = 1000

_EPS = _E // 16
_TWS = 21
_TW = 1 << _TWS
_TBL = 1 << 20
_NB = 48
_GRP = 12
_NGRP = _NB // _GRP
_BCAP = 64
_BSTRIDE = 16 * _BCAP
_TSIZE = _TBL + 1024
_DEGSZ = 12032

_mesh = plsc.VectorSubcoreMesh(core_axis_name="c", subcore_axis_name="s")
_scparams = pltpu.CompilerParams(needs_layout_passes=False,
                                 use_tc_tiling_on_sc=False)


def _make_dots():
  @functools.partial(
      pl.kernel, mesh=_mesh, compiler_params=_scparams,
      out_type=jax.ShapeDtypeStruct((_E,), jnp.float32),
      scratch_types=[
          pltpu.VMEM((_EPT,), jnp.float32),
          pltpu.VMEM((_CH, _D), jnp.float32),
          pltpu.VMEM((_CH, _D), jnp.float32),
          pltpu.VMEM((_CH, _D), jnp.float32),
          pltpu.VMEM((_CH, _D), jnp.float32),
          pltpu.VMEM((2, _CH), jnp.int32),
          pltpu.VMEM((2, _CH), jnp.int32),
          pltpu.VMEM((256,), jnp.float32),
          pltpu.SemaphoreType.DMA,
          pltpu.SemaphoreType.DMA,
          pltpu.SemaphoreType.DMA,
          pltpu.SemaphoreType.DMA,
          pltpu.SemaphoreType.DMA,
          pltpu.SemaphoreType.DMA,
      ],
  )
  def dots(ei_hbm, feat_hbm, w_out,
           wown, Xs0, Xs1, Xt0, Xt1, eib0, eib1, tsc,
           semei0, semei1, semxs0, semxs1, semxt0, semxt1):
    c = lax.axis_index("c")
    s = lax.axis_index("s")
    g = s * 2 + c
    iota = lax.iota(jnp.int32, 16)
    Xs = [Xs0, Xs1]
    Xt = [Xt0, Xt1]
    eib = [eib0, eib1]
    semei = [semei0, semei1]
    semxs = [semxs0, semxs1]
    semxt = [semxt0, semxt1]

    def start_ei(t, b):
      base = g * _EPT + t * _CH
      pltpu.async_copy(ei_hbm.at[:, pl.ds(base, _CH)], eib[b], semei[b])

    def wait_ei(b):
      pltpu.make_async_copy(ei_hbm.at[:, pl.ds(0, _CH)], eib[b],
                            semei[b]).wait()

    def start_g(b):
      pltpu.async_copy(feat_hbm.at[eib[b].at[0]], Xs[b], semxs[b])
      pltpu.async_copy(feat_hbm.at[eib[b].at[1]], Xt[b], semxt[b])

    def wait_g(b):
      pltpu.make_async_copy(feat_hbm.at[eib[b].at[0]], Xs[b],
                            semxs[b]).wait()
      pltpu.make_async_copy(feat_hbm.at[eib[b].at[1]], Xt[b],
                            semxt[b]).wait()

    start_ei(0, 0)
    start_ei(1, 1)
    wait_ei(0)
    start_g(0)

    @pl.loop(0, _NCH, step=2, unroll=1)
    def it(tb):
      for b in range(2):
        t = tb + b

        @pl.when(t < _NCH)
        def _():
          wait_g(b)

          @pl.when(t + 1 < _NCH)
          def _():
            wait_ei(1 - b)
            start_g(1 - b)

          svs = [eib[b][0, pl.ds(grp * 16, 16)] for grp in range(_CH // 16)]
          dvs = [eib[b][1, pl.ds(grp * 16, 16)] for grp in range(_CH // 16)]

          @pl.when(t + 2 < _NCH)
          def _():
            start_ei(t + 2, b)

          for grp in range(_CH // 16):
            for j in range(16):
              e = grp * 16 + j
              acc = Xs[b][e, pl.ds(0, 16)] * Xt[b][e, pl.ds(0, 16)]
              for i in range(1, _D // 16):
                acc = acc + (Xs[b][e, pl.ds(i * 16, 16)] *
                             Xt[b][e, pl.ds(i * 16, 16)])
              tsc[pl.ds(j * 16, 16)] = acc
            dotv = plsc.load_gather(tsc, [iota * 16])
            for col in range(1, 16):
              dotv = dotv + plsc.load_gather(tsc, [iota * 16 + col])
            keep = (dotv >= 0.0) & (svs[grp] != dvs[grp])
            wown[pl.ds(t * _CH + grp * 16, 16)] = jnp.where(keep, 1.0, 0.0)

    pltpu.sync_copy(wown, w_out.at[pl.ds(g * _EPT, _EPT)])
  return dots


def _make_count():
  @functools.partial(
      pl.kernel, mesh=_mesh, compiler_params=_scparams,
      out_type=[
          jax.ShapeDtypeStruct((_E,), jnp.float32),
          jax.ShapeDtypeStruct((2, _N), jnp.float32),
          jax.ShapeDtypeStruct((2, _N), jnp.float32),
      ],
      scratch_types=[
          pltpu.VMEM((_EPS,), jnp.int32),
          pltpu.VMEM(((_GRP + 1) * _BSTRIDE,), jnp.int32),
          pltpu.VMEM(((_GRP + 1) * 16,), jnp.int32),
          pltpu.VMEM((_EPT + 16,), jnp.float32),
          pltpu.VMEM((2, 4000), jnp.int32),
          pltpu.VMEM((_BSTRIDE,), jnp.int32),
          pltpu.VMEM((_BSTRIDE,), jnp.int32),
          pltpu.VMEM((_BSTRIDE,), jnp.int32),
          pltpu.VMEM((_BSTRIDE,), jnp.float32),
          pltpu.VMEM((_BSTRIDE,), jnp.float32),
          pltpu.VMEM((_BSTRIDE,), jnp.float32),
          pltpu.VMEM((1024,), jnp.float32),
          pltpu.VMEM_SHARED((_TSIZE,), jnp.float32),
          pltpu.VMEM_SHARED((_DEGSZ,), jnp.float32),
          pltpu.VMEM_SHARED((_DEGSZ,), jnp.float32),
      ],
  )
  def count(ei_hbm, wv_hbm, w_out, dego_out, degi_out,
            codes, bpos, bcnt, wown, eib,
            cidx, dsrc, ddst, dwv, cnts, addv, zeros,
            table, dego_sp, degi_sp):
    c = lax.axis_index("c")
    s = lax.axis_index("s")
    g = s * 2 + c
    iota = lax.iota(jnp.int32, 16)
    fzero = jnp.zeros((16,), jnp.float32)
    fone = jnp.ones((16,), jnp.float32)

    def zf(i, _):
      zeros[pl.ds(i * 16, 16)] = fzero
      return 0
    lax.fori_loop(0, 1024 // 16, zf, 0)
    TPT = _TSIZE // 16
    def zt(q, _):
      pltpu.sync_copy(zeros, table.at[pl.ds(s * TPT + q * 1024, 1024)])
      return 0
    lax.fori_loop(0, TPT // 1024, zt, 0)
    rem = TPT % 1024
    pltpu.sync_copy(zeros.at[pl.ds(0, rem)],
                    table.at[pl.ds(s * TPT + (TPT // 1024) * 1024, rem)])
    DPT = _DEGSZ // 16
    pltpu.sync_copy(zeros.at[pl.ds(0, DPT)], dego_sp.at[pl.ds(s * DPT, DPT)])
    pltpu.sync_copy(zeros.at[pl.ds(0, DPT)], degi_sp.at[pl.ds(s * DPT, DPT)])
    pltpu.sync_copy(wv_hbm.at[pl.ds(g * _EPT, _EPT)],
                    wown.at[pl.ds(0, _EPT)])

    def cchunk(t, _):
      base = s * _EPS + t * 4000
      pltpu.sync_copy(ei_hbm.at[:, pl.ds(base, 4000)], eib)

      def cvec(j, _):
        sv = eib[0, pl.ds(j * 16, 16)]
        dv = eib[1, pl.ds(j * 16, 16)]
        codes[pl.ds(t * 4000 + j * 16, 16)] = sv * _N + dv
        return 0
      lax.fori_loop(0, 4000 // 16, cvec, 0)
      return 0
    lax.fori_loop(0, _EPS // 4000, cchunk, 0)

    def group(grp, _):
      for k in range(_GRP + 1):
        bcnt[pl.ds(k * 16, 16)] = jnp.zeros((16,), jnp.int32)

      @pl.loop(0, _EPS // 16, unroll=1)
      def bkt(j):
        cv = codes[pl.ds(j * 16, 16)]
        b = lax.shift_right_logical(cv, _TWS)
        bg = b - grp * _GRP
        bl = jnp.where((bg >= 0) & (bg < _GRP), bg, _GRP)
        ci = bl * 16 + iota
        cur = plsc.load_gather(bcnt, [ci])
        curc = jnp.minimum(cur, _BCAP - 1)
        addr = bl * _BSTRIDE + iota * _BCAP + curc
        plsc.store_scatter(bpos, [addr], j * 16 + iota)
        plsc.store_scatter(bcnt, [ci], cur + 1)
      plsc.subcore_barrier()

      def cpass(pp, _):
        kcnt = plsc.load_gather(bcnt, [pp * 16 + iota])
        lo = (grp * _GRP + pp) * _TW

        @pl.loop(0, _BCAP, unroll=1)
        def mk_lists(k):
          pos = plsc.load_gather(bpos, [pp * _BSTRIDE + iota * _BCAP + k])
          posc = jnp.clip(pos, 0, _EPS - 1)
          cv = plsc.load_gather(codes, [posc])
          live = kcnt > k
          dump = _TBL + iota * _BCAP + k
          cidx[pl.ds(k * 16, 16)] = jnp.where(
              live, lax.shift_right_logical(cv - lo, 1), dump)
          addv[pl.ds(k * 16, 16)] = jnp.where(
              (cv & 1) == 0, 1.0, 65536.0)
          own = (posc >= c * _EPT) & (posc < (c + 1) * _EPT)
          lived = live & own
          s0 = (cv.astype(jnp.float32) * (1.0 / _N)).astype(jnp.int32)
          r0 = cv - s0 * _N
          s1 = jnp.where(r0 < 0, s0 - 1, jnp.where(r0 >= _N, s0 + 1, s0))
          d1 = cv - s1 * _N
          ddump = _N + iota * _BCAP + k
          dsrc[pl.ds(k * 16, 16)] = jnp.where(lived, s1, ddump)
          ddst[pl.ds(k * 16, 16)] = jnp.where(lived, d1, ddump)
        pltpu.sync_copy(addv, table.at[cidx], add=True)
        plsc.subcore_barrier()
        pltpu.sync_copy(table.at[cidx], cnts)

        @pl.loop(0, _BCAP, unroll=1)
        def upd_w(k):
          pos = plsc.load_gather(bpos, [pp * _BSTRIDE + iota * _BCAP + k])
          posc = jnp.clip(pos, 0, _EPS - 1)
          cv = plsc.load_gather(codes, [posc])
          live = kcnt > k
          own = (posc >= c * _EPT) & (posc < (c + 1) * _EPT)
          lived = live & own
          lown = jnp.where(lived, jnp.clip(posc - c * _EPT, 0, _EPT - 1),
                           _EPT + iota)
          wold = plsc.load_gather(wown, [lown])
          v = cnts[pl.ds(k * 16, 16)]
          hi = (v * (1.0 / 65536.0)).astype(jnp.int32).astype(jnp.float32)
          lo16 = v - hi * 65536.0
          m = jnp.maximum(jnp.where((cv & 1) == 0, lo16, hi), 1.0)
          wnew = wold * (1.0 / m)
          plsc.store_scatter(wown, [lown], wnew)
          dwv[pl.ds(k * 16, 16)] = jnp.where(lived, wnew, 0.0)
        pltpu.sync_copy(dwv, dego_sp.at[dsrc], add=True)
        pltpu.sync_copy(dwv, degi_sp.at[ddst], add=True)
        plsc.subcore_barrier()
        pltpu.sync_copy(zeros.at[pl.ds(0, _BSTRIDE)], table.at[cidx])
        plsc.subcore_barrier()
        return 0
      lax.fori_loop(0, _GRP, cpass, 0)
      return 0
    lax.fori_loop(0, _NGRP, group, 0)

    pltpu.sync_copy(wown.at[pl.ds(0, _EPT)], w_out.at[pl.ds(g * _EPT, _EPT)])

    @pl.when(s == 0)
    def _():
      pltpu.sync_copy(dego_sp.at[pl.ds(0, _N)], dego_out.at[c])
      pltpu.sync_copy(degi_sp.at[pl.ds(0, _N)], degi_out.at[c])
  return count


def _make_agg(F):
  @functools.partial(
      pl.kernel, mesh=_mesh, compiler_params=_scparams,
      out_type=jax.ShapeDtypeStruct((2, _N, F), jnp.float32),
      scratch_types=[
          pltpu.VMEM((2, _CH), jnp.int32),
          pltpu.VMEM((2, _CH), jnp.int32),
          pltpu.VMEM((_CH,), jnp.float32),
          pltpu.VMEM((_CH,), jnp.float32),
          pltpu.VMEM((_CH, F), jnp.float32),
          pltpu.VMEM((_CH, F), jnp.float32),
          pltpu.VMEM((_RPT, F), jnp.float32),
          pltpu.VMEM_SHARED((_N, F), jnp.float32),
          pltpu.SemaphoreType.DMA,
          pltpu.SemaphoreType.DMA,
          pltpu.SemaphoreType.DMA,
          pltpu.SemaphoreType.DMA,
          pltpu.SemaphoreType.DMA,
          pltpu.SemaphoreType.DMA,
      ],
  )
  def agg(ei_hbm, w_hbm, m_hbm, out_hbm,
          eib0, eib1, wch0, wch1, rows0, rows1, stage, table,
          semei0, semei1, semw0, semw1, semg0, semg1):
    c = lax.axis_index("c")
    s = lax.axis_index("s")
    g = s * 2 + c
    eib = [eib0, eib1]
    wch = [wch0, wch1]
    rows = [rows0, rows1]
    semei = [semei0, semei1]
    semw = [semw0, semw1]
    semg = [semg0, semg1]

    zero = jnp.zeros((16,), jnp.float32)
    def zrow(r, _):
      for i in range(F // 16):
        stage[r, pl.ds(i * 16, 16)] = zero
      return 0
    lax.fori_loop(0, _RPT, zrow, 0)

    @pl.when(s < _N // _RPT)
    def _():
      pltpu.sync_copy(stage, table.at[pl.ds(s * _RPT, _RPT)])

    def start_eiw(t, b):
      base = g * _EPT + t * _CH
      pltpu.async_copy(ei_hbm.at[:, pl.ds(base, _CH)], eib[b], semei[b])
      pltpu.async_copy(w_hbm.at[pl.ds(base, _CH)], wch[b], semw[b])

    def wait_ei(b):
      pltpu.make_async_copy(ei_hbm.at[:, pl.ds(0, _CH)], eib[b],
                            semei[b]).wait()

    def wait_w(b):
      pltpu.make_async_copy(w_hbm.at[pl.ds(0, _CH)], wch[b], semw[b]).wait()

    def start_g(b):
      pltpu.async_copy(m_hbm.at[eib[b].at[0]], rows[b], semg[b])

    def wait_g(b):
      pltpu.make_async_copy(m_hbm.at[eib[b].at[0]], rows[b], semg[b]).wait()

    start_eiw(0, 0)
    start_eiw(1, 1)
    plsc.subcore_barrier()
    wait_ei(0)
    start_g(0)

    @pl.loop(0, _NCH, step=2, unroll=1)
    def it(tb):
      for b in range(2):
        t = tb + b

        @pl.when(t < _NCH)
        def _():
          wait_g(b)
          wait_w(b)

          @pl.when(t + 1 < _NCH)
          def _():
            wait_ei(1 - b)
            start_g(1 - b)

          for e0 in range(0, _CH, 16):
            wgrp = wch[b][pl.ds(e0, 16)]
            for j in range(16):
              e = e0 + j
              wv = jnp.broadcast_to(wgrp[j], (16,))
              for i in range(F // 16):
                rows[b][e, pl.ds(i * 16, 16)] = (
                    rows[b][e, pl.ds(i * 16, 16)] * wv)
          pltpu.sync_copy(rows[b], table.at[eib[b].at[1]], add=True)

          @pl.when(t + 2 < _NCH)
          def _():
            start_eiw(t + 2, b)

    plsc.subcore_barrier()
    @pl.when(s < _N // _RPT)
    def _():
      pltpu.sync_copy(table.at[pl.ds(s * _RPT, _RPT)],
                      out_hbm.at[c, pl.ds(s * _RPT, _RPT)])
  return agg


_dots = _make_dots()
_count = _make_count()
_agg64 = _make_agg(_H)
_agg48 = _make_agg(_CP)


def _tc_norm_mm1(dego, degi, x, W1):
  B = 1000

  def body(dego_ref, degi_ref, x_ref, w_ref, m_ref, ns_ref, nd_ref):
    do = dego_ref[:, 0:1] + dego_ref[:, 1:2] + 1.0
    di = degi_ref[:, 0:1] + degi_ref[:, 1:2] + 1.0
    ns = lax.rsqrt(do)
    nd = lax.rsqrt(di)
    ns_ref[...] = ns
    nd_ref[...] = nd
    h = x_ref[...] * ns
    m_ref[...] = jnp.dot(h, w_ref[...], preferred_element_type=jnp.float32)

  return pl.pallas_call(
      body,
      grid=(_N // B,),
      in_specs=[
          pl.BlockSpec((B, 2), lambda i: (i, 0)),
          pl.BlockSpec((B, 2), lambda i: (i, 0)),
          pl.BlockSpec((B, _D), lambda i: (i, 0)),
          pl.BlockSpec((_D, _H), lambda i: (0, 0)),
      ],
      out_specs=[
          pl.BlockSpec((B, _H), lambda i: (i, 0)),
          pl.BlockSpec((B, 1), lambda i: (i, 0)),
          pl.BlockSpec((B, 1), lambda i: (i, 0)),
      ],
      out_shape=[
          jax.ShapeDtypeStruct((_N, _H), jnp.float32),
          jax.ShapeDtypeStruct((_N, 1), jnp.float32),
          jax.ShapeDtypeStruct((_N, 1), jnp.float32),
      ],
  )(dego, degi, x, W1)


def _tc_layer1_mm2(aggp, m1, ns, nd, b1, W2p):
  def body(a_ref, m_ref, ns_ref, nd_ref, b_ref, w_ref, m2_ref):
    agg = a_ref[0] + a_ref[1] + m_ref[...]
    o1 = jnp.maximum(agg * nd_ref[...] + b_ref[...], 0.0)
    h = o1 * ns_ref[...]
    m2_ref[...] = jnp.dot(h, w_ref[...], preferred_element_type=jnp.float32)

  B = 1000
  return pl.pallas_call(
      body,
      grid=(_N // B,),
      in_specs=[
          pl.BlockSpec((2, B, _H), lambda i: (0, i, 0)),
          pl.BlockSpec((B, _H), lambda i: (i, 0)),
          pl.BlockSpec((B, 1), lambda i: (i, 0)),
          pl.BlockSpec((B, 1), lambda i: (i, 0)),
          pl.BlockSpec((1, _H), lambda i: (0, 0)),
          pl.BlockSpec((_H, _CP), lambda i: (0, 0)),
      ],
      out_specs=pl.BlockSpec((B, _CP), lambda i: (i, 0)),
      out_shape=jax.ShapeDtypeStruct((_N, _CP), jnp.float32),
  )(aggp, m1, ns, nd, b1, W2p)


def _tc_final(aggp, m2, nd, b2):
  def body(a_ref, m_ref, nd_ref, b_ref, o_ref):
    agg = a_ref[0] + a_ref[1] + m_ref[...]
    o_ref[...] = agg[:, :_C] * nd_ref[...] + b_ref[...]

  B = 1000
  return pl.pallas_call(
      body,
      grid=(_N // B,),
      in_specs=[
          pl.BlockSpec((2, B, _CP), lambda i: (0, i, 0)),
          pl.BlockSpec((B, _CP), lambda i: (i, 0)),
          pl.BlockSpec((B, 1), lambda i: (i, 0)),
          pl.BlockSpec((1, _C), lambda i: (0, 0)),
      ],
      out_specs=pl.BlockSpec((B, _C), lambda i: (i, 0)),
      out_shape=jax.ShapeDtypeStruct((_N, _C), jnp.float32),
  )(aggp, m2, nd, b2)


def kernel(edge_index, features, W1, b1, W2, b2):
  ei = edge_index.astype(jnp.int32)

  wv = _dots(ei, features)
  w, dego, degi = _count(ei, wv)
  m1, ns, nd = _tc_norm_mm1(dego.T, degi.T, features, W1)
  agg1 = _agg64(ei, w, m1)
  W2p = jnp.pad(W2, ((0, 0), (0, _CP - _C)))
  m2 = _tc_layer1_mm2(agg1, m1, ns, nd, b1[None, :], W2p)
  agg2 = _agg48(ei, w, m2)
  return _tc_final(agg2, m2, nd, b2[None, :])


# speedup vs baseline: 5.9105x; 1.0174x over previous
import functools
import jax
import jax.numpy as jnp
from jax import lax
from jax.experimental import pallas as pl
from jax.experimental.pallas import tpu as pltpu
from jax.experimental.pallas import tpu_sc as plsc

_N = 10000
_E = 320000
_D = 128
_H = 64
_C = 40
_CP = 48
_TILES = 32
_EPT = _E // _TILES
_CH = 80
_NCH = _EPT // _CH
_RPT = 1000

_EPS = _E // 16
_TWS = 21
_TW = 1 << _TWS
_TBL = 1 << 20
_NB = 48
_GRP = 12
_NGRP = _NB // _GRP
_BCAP = 64
_BSTRIDE = 16 * _BCAP
_TSIZE = _TBL + 1024
_DEGSZ = 12032

_mesh = plsc.VectorSubcoreMesh(core_axis_name="c", subcore_axis_name="s")
_scparams = pltpu.CompilerParams(needs_layout_passes=False,
                                 use_tc_tiling_on_sc=False)


def _make_dots():
  @functools.partial(
      pl.kernel, mesh=_mesh, compiler_params=_scparams,
      out_type=jax.ShapeDtypeStruct((_E,), jnp.float32),
      scratch_types=[
          pltpu.VMEM((_EPT,), jnp.float32),
          pltpu.VMEM((_CH, _D), jnp.float32),
          pltpu.VMEM((_CH, _D), jnp.float32),
          pltpu.VMEM((_CH, _D), jnp.float32),
          pltpu.VMEM((_CH, _D), jnp.float32),
          pltpu.VMEM((2, _CH), jnp.int32),
          pltpu.VMEM((2, _CH), jnp.int32),
          pltpu.VMEM((256,), jnp.float32),
          pltpu.SemaphoreType.DMA,
          pltpu.SemaphoreType.DMA,
          pltpu.SemaphoreType.DMA,
          pltpu.SemaphoreType.DMA,
          pltpu.SemaphoreType.DMA,
          pltpu.SemaphoreType.DMA,
      ],
  )
  def dots(ei_hbm, feat_hbm, w_out,
           wown, Xs0, Xs1, Xt0, Xt1, eib0, eib1, tsc,
           semei0, semei1, semxs0, semxs1, semxt0, semxt1):
    c = lax.axis_index("c")
    s = lax.axis_index("s")
    g = s * 2 + c
    iota = lax.iota(jnp.int32, 16)
    Xs = [Xs0, Xs1]
    Xt = [Xt0, Xt1]
    eib = [eib0, eib1]
    semei = [semei0, semei1]
    semxs = [semxs0, semxs1]
    semxt = [semxt0, semxt1]

    def start_ei(t, b):
      base = g * _EPT + t * _CH
      pltpu.async_copy(ei_hbm.at[:, pl.ds(base, _CH)], eib[b], semei[b])

    def wait_ei(b):
      pltpu.make_async_copy(ei_hbm.at[:, pl.ds(0, _CH)], eib[b],
                            semei[b]).wait()

    def start_g(b):
      pltpu.async_copy(feat_hbm.at[eib[b].at[0]], Xs[b], semxs[b])
      pltpu.async_copy(feat_hbm.at[eib[b].at[1]], Xt[b], semxt[b])

    def wait_g(b):
      pltpu.make_async_copy(feat_hbm.at[eib[b].at[0]], Xs[b],
                            semxs[b]).wait()
      pltpu.make_async_copy(feat_hbm.at[eib[b].at[1]], Xt[b],
                            semxt[b]).wait()

    start_ei(0, 0)
    start_ei(1, 1)
    wait_ei(0)
    start_g(0)

    @pl.loop(0, _NCH, step=2, unroll=1)
    def it(tb):
      for b in range(2):
        t = tb + b

        @pl.when(t < _NCH)
        def _():
          wait_g(b)

          @pl.when(t + 1 < _NCH)
          def _():
            wait_ei(1 - b)
            start_g(1 - b)

          svs = [eib[b][0, pl.ds(grp * 16, 16)] for grp in range(_CH // 16)]
          dvs = [eib[b][1, pl.ds(grp * 16, 16)] for grp in range(_CH // 16)]

          @pl.when(t + 2 < _NCH)
          def _():
            start_ei(t + 2, b)

          for grp in range(_CH // 16):
            for j in range(16):
              e = grp * 16 + j
              acc = Xs[b][e, pl.ds(0, 16)] * Xt[b][e, pl.ds(0, 16)]
              for i in range(1, _D // 16):
                acc = acc + (Xs[b][e, pl.ds(i * 16, 16)] *
                             Xt[b][e, pl.ds(i * 16, 16)])
              tsc[pl.ds(j * 16, 16)] = acc
            dotv = plsc.load_gather(tsc, [iota * 16])
            for col in range(1, 16):
              dotv = dotv + plsc.load_gather(tsc, [iota * 16 + col])
            keep = (dotv >= 0.0) & (svs[grp] != dvs[grp])
            wown[pl.ds(t * _CH + grp * 16, 16)] = jnp.where(keep, 1.0, 0.0)

    pltpu.sync_copy(wown, w_out.at[pl.ds(g * _EPT, _EPT)])
  return dots


def _make_count():
  @functools.partial(
      pl.kernel, mesh=_mesh, compiler_params=_scparams,
      out_type=[
          jax.ShapeDtypeStruct((_E,), jnp.float32),
          jax.ShapeDtypeStruct((2, _N), jnp.float32),
          jax.ShapeDtypeStruct((2, _N), jnp.float32),
      ],
      scratch_types=[
          pltpu.VMEM((_EPS,), jnp.int32),
          pltpu.VMEM(((_GRP + 1) * _BSTRIDE,), jnp.int32),
          pltpu.VMEM(((_GRP + 1) * 16,), jnp.int32),
          pltpu.VMEM((_EPT + 16,), jnp.float32),
          pltpu.VMEM((2, 4000), jnp.int32),
          pltpu.VMEM((_BSTRIDE,), jnp.int32),
          pltpu.VMEM((_BSTRIDE,), jnp.int32),
          pltpu.VMEM((_BSTRIDE,), jnp.int32),
          pltpu.VMEM((_BSTRIDE,), jnp.float32),
          pltpu.VMEM((_BSTRIDE,), jnp.float32),
          pltpu.VMEM((_BSTRIDE,), jnp.float32),
          pltpu.VMEM((_BSTRIDE,), jnp.float32),
          pltpu.VMEM((1024,), jnp.float32),
          pltpu.VMEM_SHARED((_TSIZE,), jnp.float32),
          pltpu.VMEM_SHARED((_DEGSZ,), jnp.float32),
          pltpu.VMEM_SHARED((_DEGSZ,), jnp.float32),
      ],
  )
  def count(ei_hbm, wv_hbm, w_out, dego_out, degi_out,
            codes, bpos, bcnt, wown, eib,
            cidx, dsrc, ddst, dwv, cnts, addv, negv, zeros,
            table, dego_sp, degi_sp):
    c = lax.axis_index("c")
    s = lax.axis_index("s")
    g = s * 2 + c
    iota = lax.iota(jnp.int32, 16)
    fzero = jnp.zeros((16,), jnp.float32)
    fone = jnp.ones((16,), jnp.float32)

    def zf(i, _):
      zeros[pl.ds(i * 16, 16)] = fzero
      return 0
    lax.fori_loop(0, 1024 // 16, zf, 0)
    TPT = _TSIZE // 16
    def zt(q, _):
      pltpu.sync_copy(zeros, table.at[pl.ds(s * TPT + q * 1024, 1024)])
      return 0
    lax.fori_loop(0, TPT // 1024, zt, 0)
    rem = TPT % 1024
    pltpu.sync_copy(zeros.at[pl.ds(0, rem)],
                    table.at[pl.ds(s * TPT + (TPT // 1024) * 1024, rem)])
    DPT = _DEGSZ // 16
    pltpu.sync_copy(zeros.at[pl.ds(0, DPT)], dego_sp.at[pl.ds(s * DPT, DPT)])
    pltpu.sync_copy(zeros.at[pl.ds(0, DPT)], degi_sp.at[pl.ds(s * DPT, DPT)])
    pltpu.sync_copy(wv_hbm.at[pl.ds(g * _EPT, _EPT)],
                    wown.at[pl.ds(0, _EPT)])

    def cchunk(t, _):
      base = s * _EPS + t * 4000
      pltpu.sync_copy(ei_hbm.at[:, pl.ds(base, 4000)], eib)

      def cvec(j, _):
        sv = eib[0, pl.ds(j * 16, 16)]
        dv = eib[1, pl.ds(j * 16, 16)]
        codes[pl.ds(t * 4000 + j * 16, 16)] = sv * _N + dv
        return 0
      lax.fori_loop(0, 4000 // 16, cvec, 0)
      return 0
    lax.fori_loop(0, _EPS // 4000, cchunk, 0)

    def group(grp, _):
      for k in range(_GRP + 1):
        bcnt[pl.ds(k * 16, 16)] = jnp.zeros((16,), jnp.int32)

      @pl.loop(0, _EPS // 16, unroll=1)
      def bkt(j):
        cv = codes[pl.ds(j * 16, 16)]
        b = lax.shift_right_logical(cv, _TWS)
        bg = b - grp * _GRP
        bl = jnp.where((bg >= 0) & (bg < _GRP), bg, _GRP)
        ci = bl * 16 + iota
        cur = plsc.load_gather(bcnt, [ci])
        curc = jnp.minimum(cur, _BCAP - 1)
        addr = bl * _BSTRIDE + iota * _BCAP + curc
        plsc.store_scatter(bpos, [addr], j * 16 + iota)
        plsc.store_scatter(bcnt, [ci], cur + 1)
      plsc.subcore_barrier()

      def cpass(pp, _):
        kcnt = plsc.load_gather(bcnt, [pp * 16 + iota])
        lo = (grp * _GRP + pp) * _TW

        @pl.loop(0, _BCAP, unroll=1)
        def mk_lists(k):
          pos = plsc.load_gather(bpos, [pp * _BSTRIDE + iota * _BCAP + k])
          posc = jnp.clip(pos, 0, _EPS - 1)
          cv = plsc.load_gather(codes, [posc])
          live = kcnt > k
          dump = _TBL + iota * _BCAP + k
          cidx[pl.ds(k * 16, 16)] = jnp.where(
              live, lax.shift_right_logical(cv - lo, 1), dump)
          av = jnp.where((cv & 1) == 0, 1.0, 65536.0)
          addv[pl.ds(k * 16, 16)] = av
          negv[pl.ds(k * 16, 16)] = -av
          own = (posc >= c * _EPT) & (posc < (c + 1) * _EPT)
          lived = live & own
          s0 = (cv.astype(jnp.float32) * (1.0 / _N)).astype(jnp.int32)
          r0 = cv - s0 * _N
          s1 = jnp.where(r0 < 0, s0 - 1, jnp.where(r0 >= _N, s0 + 1, s0))
          d1 = cv - s1 * _N
          ddump = _N + iota * _BCAP + k
          dsrc[pl.ds(k * 16, 16)] = jnp.where(lived, s1, ddump)
          ddst[pl.ds(k * 16, 16)] = jnp.where(lived, d1, ddump)
        pltpu.sync_copy(addv, table.at[cidx], add=True)
        plsc.subcore_barrier()
        pltpu.sync_copy(table.at[cidx], cnts)

        @pl.loop(0, _BCAP, unroll=1)
        def upd_w(k):
          pos = plsc.load_gather(bpos, [pp * _BSTRIDE + iota * _BCAP + k])
          posc = jnp.clip(pos, 0, _EPS - 1)
          cv = plsc.load_gather(codes, [posc])
          live = kcnt > k
          own = (posc >= c * _EPT) & (posc < (c + 1) * _EPT)
          lived = live & own
          lown = jnp.where(lived, jnp.clip(posc - c * _EPT, 0, _EPT - 1),
                           _EPT + iota)
          wold = plsc.load_gather(wown, [lown])
          v = cnts[pl.ds(k * 16, 16)]
          hi = (v * (1.0 / 65536.0)).astype(jnp.int32).astype(jnp.float32)
          lo16 = v - hi * 65536.0
          m = jnp.maximum(jnp.where((cv & 1) == 0, lo16, hi), 1.0)
          wnew = wold * (1.0 / m)
          plsc.store_scatter(wown, [lown], wnew)
          dwv[pl.ds(k * 16, 16)] = jnp.where(lived, wnew, 0.0)
        pltpu.sync_copy(dwv, dego_sp.at[dsrc], add=True)
        pltpu.sync_copy(dwv, degi_sp.at[ddst], add=True)
        plsc.subcore_barrier()
        pltpu.sync_copy(negv, table.at[cidx], add=True)
        return 0
      lax.fori_loop(0, _GRP, cpass, 0)
      return 0
    lax.fori_loop(0, _NGRP, group, 0)

    pltpu.sync_copy(wown.at[pl.ds(0, _EPT)], w_out.at[pl.ds(g * _EPT, _EPT)])

    @pl.when(s == 0)
    def _():
      pltpu.sync_copy(dego_sp.at[pl.ds(0, _N)], dego_out.at[c])
      pltpu.sync_copy(degi_sp.at[pl.ds(0, _N)], degi_out.at[c])
  return count


def _make_agg(F):
  @functools.partial(
      pl.kernel, mesh=_mesh, compiler_params=_scparams,
      out_type=jax.ShapeDtypeStruct((2, _N, F), jnp.float32),
      scratch_types=[
          pltpu.VMEM((2, _CH), jnp.int32),
          pltpu.VMEM((2, _CH), jnp.int32),
          pltpu.VMEM((_CH,), jnp.float32),
          pltpu.VMEM((_CH,), jnp.float32),
          pltpu.VMEM((_CH, F), jnp.float32),
          pltpu.VMEM((_CH, F), jnp.float32),
          pltpu.VMEM((_RPT, F), jnp.float32),
          pltpu.VMEM_SHARED((_N, F), jnp.float32),
          pltpu.SemaphoreType.DMA,
          pltpu.SemaphoreType.DMA,
          pltpu.SemaphoreType.DMA,
          pltpu.SemaphoreType.DMA,
          pltpu.SemaphoreType.DMA,
          pltpu.SemaphoreType.DMA,
      ],
  )
  def agg(ei_hbm, w_hbm, m_hbm, out_hbm,
          eib0, eib1, wch0, wch1, rows0, rows1, stage, table,
          semei0, semei1, semw0, semw1, semg0, semg1):
    c = lax.axis_index("c")
    s = lax.axis_index("s")
    g = s * 2 + c
    eib = [eib0, eib1]
    wch = [wch0, wch1]
    rows = [rows0, rows1]
    semei = [semei0, semei1]
    semw = [semw0, semw1]
    semg = [semg0, semg1]

    zero = jnp.zeros((16,), jnp.float32)
    def zrow(r, _):
      for i in range(F // 16):
        stage[r, pl.ds(i * 16, 16)] = zero
      return 0
    lax.fori_loop(0, _RPT, zrow, 0)

    @pl.when(s < _N // _RPT)
    def _():
      pltpu.sync_copy(stage, table.at[pl.ds(s * _RPT, _RPT)])

    def start_eiw(t, b):
      base = g * _EPT + t * _CH
      pltpu.async_copy(ei_hbm.at[:, pl.ds(base, _CH)], eib[b], semei[b])
      pltpu.async_copy(w_hbm.at[pl.ds(base, _CH)], wch[b], semw[b])

    def wait_ei(b):
      pltpu.make_async_copy(ei_hbm.at[:, pl.ds(0, _CH)], eib[b],
                            semei[b]).wait()

    def wait_w(b):
      pltpu.make_async_copy(w_hbm.at[pl.ds(0, _CH)], wch[b], semw[b]).wait()

    def start_g(b):
      pltpu.async_copy(m_hbm.at[eib[b].at[0]], rows[b], semg[b])

    def wait_g(b):
      pltpu.make_async_copy(m_hbm.at[eib[b].at[0]], rows[b], semg[b]).wait()

    start_eiw(0, 0)
    start_eiw(1, 1)
    plsc.subcore_barrier()
    wait_ei(0)
    start_g(0)

    @pl.loop(0, _NCH, step=2, unroll=1)
    def it(tb):
      for b in range(2):
        t = tb + b

        @pl.when(t < _NCH)
        def _():
          wait_g(b)
          wait_w(b)

          @pl.when(t + 1 < _NCH)
          def _():
            wait_ei(1 - b)
            start_g(1 - b)

          for e0 in range(0, _CH, 16):
            wgrp = wch[b][pl.ds(e0, 16)]
            for j in range(16):
              e = e0 + j
              wv = jnp.broadcast_to(wgrp[j], (16,))
              for i in range(F // 16):
                rows[b][e, pl.ds(i * 16, 16)] = (
                    rows[b][e, pl.ds(i * 16, 16)] * wv)
          pltpu.sync_copy(rows[b], table.at[eib[b].at[1]], add=True)

          @pl.when(t + 2 < _NCH)
          def _():
            start_eiw(t + 2, b)

    plsc.subcore_barrier()
    @pl.when(s < _N // _RPT)
    def _():
      pltpu.sync_copy(table.at[pl.ds(s * _RPT, _RPT)],
                      out_hbm.at[c, pl.ds(s * _RPT, _RPT)])
  return agg


_dots = _make_dots()
_count = _make_count()
_agg64 = _make_agg(_H)
_agg48 = _make_agg(_CP)


def _tc_norm_mm1(dego, degi, x, W1):
  B = 1000

  def body(dego_ref, degi_ref, x_ref, w_ref, m_ref, ns_ref, nd_ref):
    do = dego_ref[:, 0:1] + dego_ref[:, 1:2] + 1.0
    di = degi_ref[:, 0:1] + degi_ref[:, 1:2] + 1.0
    ns = lax.rsqrt(do)
    nd = lax.rsqrt(di)
    ns_ref[...] = ns
    nd_ref[...] = nd
    h = x_ref[...] * ns
    m_ref[...] = jnp.dot(h, w_ref[...], preferred_element_type=jnp.float32)

  return pl.pallas_call(
      body,
      grid=(_N // B,),
      in_specs=[
          pl.BlockSpec((B, 2), lambda i: (i, 0)),
          pl.BlockSpec((B, 2), lambda i: (i, 0)),
          pl.BlockSpec((B, _D), lambda i: (i, 0)),
          pl.BlockSpec((_D, _H), lambda i: (0, 0)),
      ],
      out_specs=[
          pl.BlockSpec((B, _H), lambda i: (i, 0)),
          pl.BlockSpec((B, 1), lambda i: (i, 0)),
          pl.BlockSpec((B, 1), lambda i: (i, 0)),
      ],
      out_shape=[
          jax.ShapeDtypeStruct((_N, _H), jnp.float32),
          jax.ShapeDtypeStruct((_N, 1), jnp.float32),
          jax.ShapeDtypeStruct((_N, 1), jnp.float32),
      ],
  )(dego, degi, x, W1)


def _tc_layer1_mm2(aggp, m1, ns, nd, b1, W2p):
  def body(a_ref, m_ref, ns_ref, nd_ref, b_ref, w_ref, m2_ref):
    agg = a_ref[0] + a_ref[1] + m_ref[...]
    o1 = jnp.maximum(agg * nd_ref[...] + b_ref[...], 0.0)
    h = o1 * ns_ref[...]
    m2_ref[...] = jnp.dot(h, w_ref[...], preferred_element_type=jnp.float32)

  B = 1000
  return pl.pallas_call(
      body,
      grid=(_N // B,),
      in_specs=[
          pl.BlockSpec((2, B, _H), lambda i: (0, i, 0)),
          pl.BlockSpec((B, _H), lambda i: (i, 0)),
          pl.BlockSpec((B, 1), lambda i: (i, 0)),
          pl.BlockSpec((B, 1), lambda i: (i, 0)),
          pl.BlockSpec((1, _H), lambda i: (0, 0)),
          pl.BlockSpec((_H, _CP), lambda i: (0, 0)),
      ],
      out_specs=pl.BlockSpec((B, _CP), lambda i: (i, 0)),
      out_shape=jax.ShapeDtypeStruct((_N, _CP), jnp.float32),
  )(aggp, m1, ns, nd, b1, W2p)


def _tc_final(aggp, m2, nd, b2):
  def body(a_ref, m_ref, nd_ref, b_ref, o_ref):
    agg = a_ref[0] + a_ref[1] + m_ref[...]
    o_ref[...] = agg[:, :_C] * nd_ref[...] + b_ref[...]

  B = 1000
  return pl.pallas_call(
      body,
      grid=(_N // B,),
      in_specs=[
          pl.BlockSpec((2, B, _CP), lambda i: (0, i, 0)),
          pl.BlockSpec((B, _CP), lambda i: (i, 0)),
          pl.BlockSpec((B, 1), lambda i: (i, 0)),
          pl.BlockSpec((1, _C), lambda i: (0, 0)),
      ],
      out_specs=pl.BlockSpec((B, _C), lambda i: (i, 0)),
      out_shape=jax.ShapeDtypeStruct((_N, _C), jnp.float32),
  )(aggp, m2, nd, b2)


def kernel(edge_index, features, W1, b1, W2, b2):
  ei = edge_index.astype(jnp.int32)

  wv = _dots(ei, features)
  w, dego, degi = _count(ei, wv)
  m1, ns, nd = _tc_norm_mm1(dego.T, degi.T, features, W1)
  agg1 = _agg64(ei, w, m1)
  W2p = jnp.pad(W2, ((0, 0), (0, _CP - _C)))
  m2 = _tc_layer1_mm2(agg1, m1, ns, nd, b1[None, :], W2p)
  agg2 = _agg48(ei, w, m2)
  return _tc_final(agg2, m2, nd, b2[None, :])
